# baseline (speedup 1.0000x reference)
import functools

import jax
import jax.numpy as jnp
from jax import lax
from jax.experimental import pallas as pl
from jax.experimental.pallas import tpu as pltpu
from jax.experimental.pallas import tpu_sc as plsc

_N = 10000
_NP = 10240
_E = 320000
_D = 128
_F_IN = 128
_HID = 64
_NG = 64
_BN_EPS = 1e-5

_NC = 2
_NS = 16
_NW = _NC * _NS
_CHUNK = 80
_NB = 4
_NCHUNK = _E // _NW // _CHUNK
_ITERS = _NCHUNK // _NB
_TAIL = _NCHUNK - _ITERS * _NB


def _agg_kernel():
    per_w = _E // _NW
    rows_per_sub = _NP // _NS
    mesh = plsc.VectorSubcoreMesh(core_axis_name="c", subcore_axis_name="s")

    @functools.partial(
        pl.kernel,
        mesh=mesh,
        out_type=jax.ShapeDtypeStruct((_NC, _NP, _D), jnp.float32),
        scratch_types=(
            [pltpu.VMEM((_CHUNK,), jnp.int32) for _ in range(2 * _NB)]
            + [
                pltpu.VMEM((_NB, _CHUNK, _D), jnp.float32),
                pltpu.VMEM_SHARED((_NP, _D), jnp.float32),
                pltpu.SemaphoreType.DMA((_NB,)),
                pltpu.SemaphoreType.DMA((_NB,)),
                pltpu.SemaphoreType.DMA((_NB,)),
            ]
        ),
    )
    def agg(h_hbm, src_hbm, dst_hbm, zeros_hbm, out_hbm, *scr):
        srcbs = list(scr[:_NB])
        dstbs = list(scr[_NB:2 * _NB])
        rows_v, acc_sh, gsem, ssem, isem = scr[2 * _NB:]
        cid = lax.axis_index("c")
        sid = lax.axis_index("s")
        wid = sid * _NC + cid
        r0 = sid * rows_per_sub
        base = wid * per_w

        @pl.when(cid == 0)
        def _():
            pltpu.sync_copy(h_hbm.at[pl.ds(r0, rows_per_sub)],
                            acc_sh.at[pl.ds(r0, rows_per_sub)])

        @pl.when(cid != 0)
        def _():
            pltpu.sync_copy(zeros_hbm.at[pl.ds(r0, rows_per_sub)],
                            acc_sh.at[pl.ds(r0, rows_per_sub)])

        plsc.subcore_barrier()

        def it(t, carry):
            ids, gds = [], []
            for b in range(_NB):
                j = t * _NB + b

                @pl.when(t > 0)
                def _(b=b):
                    pltpu.make_async_copy(
                        rows_v.at[b], acc_sh.at[dstbs[b]],
                        ssem.at[b]).wait()

                off = base + j * _CHUNK
                ids.append(pltpu.async_copy(
                    src_hbm.at[pl.ds(off, _CHUNK)], srcbs[b], isem.at[b]))
                pltpu.async_copy(
                    dst_hbm.at[pl.ds(off, _CHUNK)], dstbs[b], isem.at[b])
            for b in range(_NB):
                ids[b].wait()
                pltpu.make_async_copy(
                    dst_hbm.at[pl.ds(base, _CHUNK)], dstbs[b],
                    isem.at[b]).wait()
                gds.append(pltpu.async_copy(
                    h_hbm.at[srcbs[b]], rows_v.at[b], gsem.at[b]))
            for b in range(_NB):
                gds[b].wait()
                pltpu.async_copy(rows_v.at[b], acc_sh.at[dstbs[b]],
                                 ssem.at[b], add=True)
            return carry

        lax.fori_loop(0, _ITERS, it, 0)

        for b in range(_NB):
            pltpu.make_async_copy(rows_v.at[b], acc_sh.at[dstbs[b]],
                                  ssem.at[b]).wait()
        for k in range(_TAIL):
            j = _ITERS * _NB + k
            off = base + j * _CHUNK
            pltpu.sync_copy(src_hbm.at[pl.ds(off, _CHUNK)], srcbs[k])
            pltpu.sync_copy(dst_hbm.at[pl.ds(off, _CHUNK)], dstbs[k])
            pltpu.async_copy(h_hbm.at[srcbs[k]], rows_v.at[k],
                             gsem.at[k]).wait()
            pltpu.async_copy(rows_v.at[k], acc_sh.at[dstbs[k]],
                             ssem.at[k], add=True)
        for k in range(_TAIL):
            pltpu.make_async_copy(rows_v.at[k], acc_sh.at[dstbs[k]],
                                  ssem.at[k]).wait()

        plsc.subcore_barrier()
        pltpu.sync_copy(acc_sh.at[pl.ds(r0, rows_per_sub)],
                        out_hbm.at[cid, pl.ds(r0, rows_per_sub)])

    return agg


_AGG = _agg_kernel()


def _mlp_layer(z2, w1, b1, w2, b2, gamma, beta):
    blk = 2048
    grid = _NP // blk

    def body(z_ref, w1_ref, b1_ref, w2_ref, b2_ref, g_ref, bt_ref, o_ref):
        z = z_ref[0] + z_ref[1]
        a = jnp.maximum(
            lax.dot(z, w1_ref[...], preferred_element_type=jnp.float32, precision=lax.Precision.HIGHEST)
            + b1_ref[...], 0.0)
        zz = (lax.dot(a, w2_ref[...], preferred_element_type=jnp.float32, precision=lax.Precision.HIGHEST)
              + b2_ref[...])
        scale = g_ref[...] * lax.rsqrt(jnp.float32(1.0 + _BN_EPS))
        res = jnp.maximum(zz * scale + bt_ref[...], 0.0)
        o_ref[...] = jnp.concatenate([res, jnp.zeros_like(res)], axis=1)

    return pl.pallas_call(
        body,
        grid=(grid,),
        in_specs=[
            pl.BlockSpec((_NC, blk, _D), lambda i: (0, i, 0)),
            pl.BlockSpec((_D, _HID), lambda i: (0, 0)),
            pl.BlockSpec((1, _HID), lambda i: (0, 0)),
            pl.BlockSpec((_HID, _HID), lambda i: (0, 0)),
            pl.BlockSpec((1, _HID), lambda i: (0, 0)),
            pl.BlockSpec((1, _HID), lambda i: (0, 0)),
            pl.BlockSpec((1, _HID), lambda i: (0, 0)),
        ],
        out_specs=pl.BlockSpec((blk, _D), lambda i: (i, 0)),
        out_shape=jax.ShapeDtypeStruct((_NP, _D), jnp.float32),
    )(z2, w1, b1.reshape(1, _HID), w2, b2.reshape(1, _HID),
      gamma.reshape(1, _HID), beta.reshape(1, _HID))


def _pool_head(h, batch_row, w1, b1, w2, b2):
    def body(h_ref, b_ref, w1_ref, b1_ref, w2_ref, b2_ref, o_ref):
        gids = lax.broadcasted_iota(jnp.int32, (_NG, _NP), 0)
        oh = (b_ref[...] == gids).astype(jnp.float32)
        sums = lax.dot(oh, h_ref[...], preferred_element_type=jnp.float32, precision=lax.Precision.HIGHEST)
        counts = jnp.sum(oh, axis=1, keepdims=True)
        pooled = sums / jnp.maximum(counts, 1.0)
        a = jnp.maximum(
            lax.dot(pooled, w1_ref[...], preferred_element_type=jnp.float32, precision=lax.Precision.HIGHEST)
            + b1_ref[...], 0.0)
        o_ref[...] = (lax.dot(a, w2_ref[...],
                              preferred_element_type=jnp.float32,
                              precision=lax.Precision.HIGHEST)
                      + b2_ref[...])

    return pl.pallas_call(
        body,
        out_shape=jax.ShapeDtypeStruct((_NG, 1), jnp.float32),
    )(h, batch_row, w1, b1.reshape(1, -1), w2, b2.reshape(1, -1))


def _pad_rows(w, rows):
    return jnp.concatenate(
        [w, jnp.zeros((rows - w.shape[0], w.shape[1]), w.dtype)], axis=0)


def kernel(x, edge_index, edge_attr, batch, task_id, params):
    src = edge_index[0]
    dst = edge_index[1]
    pad = _NP - _N
    h = jnp.concatenate([x, jnp.zeros((pad, _F_IN), jnp.float32)], axis=0)
    zeros = jnp.zeros((_NP, _D), jnp.float32)
    for i in range(3):
        z2 = _AGG(h, src, dst, zeros)
        p = params['gin'][i]
        bn = params['bn'][i]
        h = _mlp_layer(z2, _pad_rows(p['w1'], _D), p['b1'], p['w2'], p['b2'],
                       bn['gamma'], bn['beta'])
    batch_row = jnp.concatenate(
        [batch, jnp.full((pad,), _NG, jnp.int32)]).reshape(1, _NP)
    hsel = jax.tree_util.tree_map(
        lambda a, b: jnp.where(task_id == 0, a, b),
        params['heads'][0], params['heads'][1],
    )
    return _pool_head(h, batch_row,
                      _pad_rows(hsel['w1'], _D), hsel['b1'],
                      hsel['w2'], hsel['b2'])


# speedup vs baseline: 1.1720x; 1.1720x over previous
import functools

import jax
import jax.numpy as jnp
from jax import lax
from jax.experimental import pallas as pl
from jax.experimental.pallas import tpu as pltpu
from jax.experimental.pallas import tpu_sc as plsc

_N = 10000
_NP = 10240
_E = 320000
_D = 128
_F_IN = 128
_HID = 64
_NG = 64
_BN_EPS = 1e-5

_NC = 2
_NS = 16
_NW = _NC * _NS
_CHUNK = 80
_NB = 3
_NCHUNK = _E // _NW // _CHUNK
_ITERS = _NCHUNK // _NB
_TAIL = _NCHUNK - _ITERS * _NB


def _agg_kernel():
    per_w = _E // _NW
    rows_per_sub = _NP // _NS
    mesh = plsc.VectorSubcoreMesh(core_axis_name="c", subcore_axis_name="s")

    @functools.partial(
        pl.kernel,
        mesh=mesh,
        out_type=jax.ShapeDtypeStruct((_NC, _NP, _D), jnp.float32),
        scratch_types=[
            pltpu.VMEM((per_w,), jnp.int32),
            pltpu.VMEM((_CHUNK,), jnp.int32),
            pltpu.VMEM((_CHUNK,), jnp.int32),
            pltpu.VMEM((_CHUNK,), jnp.int32),
            pltpu.VMEM((_NB, _CHUNK, _D), jnp.float32),
            pltpu.VMEM_SHARED((_NP, _D), jnp.float32),
            pltpu.SemaphoreType.DMA((_NB,)),
            pltpu.SemaphoreType.DMA((_NB,)),
            pltpu.SemaphoreType.DMA((_NB,)),
        ],
    )
    def agg(h_hbm, src_hbm, dst_hbm, zeros_hbm, out_hbm,
            src1d, db0, db1, db2, rows_v, acc_sh, gsem, ssem, isem):
        dstbs = [db0, db1, db2]
        cid = lax.axis_index("c")
        sid = lax.axis_index("s")
        wid = sid * _NC + cid
        r0 = sid * rows_per_sub
        base = wid * per_w

        pltpu.sync_copy(src_hbm.at[pl.ds(base, per_w)], src1d)

        @pl.when(cid == 0)
        def _():
            pltpu.sync_copy(h_hbm.at[pl.ds(r0, rows_per_sub)],
                            acc_sh.at[pl.ds(r0, rows_per_sub)])

        @pl.when(cid != 0)
        def _():
            pltpu.sync_copy(zeros_hbm.at[pl.ds(r0, rows_per_sub)],
                            acc_sh.at[pl.ds(r0, rows_per_sub)])

        plsc.subcore_barrier()

        def it(t, carry):
            ids, gds = [], []
            for b in range(_NB):
                j = t * _NB + b

                @pl.when(t > 0)
                def _(b=b):
                    pltpu.make_async_copy(
                        rows_v.at[b], acc_sh.at[dstbs[b]],
                        ssem.at[b]).wait()

                ids.append(pltpu.async_copy(
                    dst_hbm.at[pl.ds(base + j * _CHUNK, _CHUNK)],
                    dstbs[b], isem.at[b]))
                gds.append(pltpu.async_copy(
                    h_hbm.at[src1d.at[pl.ds(j * _CHUNK, _CHUNK)]],
                    rows_v.at[b], gsem.at[b]))
            for b in range(_NB):
                gds[b].wait()
                ids[b].wait()
                pltpu.async_copy(rows_v.at[b], acc_sh.at[dstbs[b]],
                                 ssem.at[b], add=True)
            return carry

        lax.fori_loop(0, _ITERS, it, 0)

        for b in range(_NB):
            pltpu.make_async_copy(rows_v.at[b], acc_sh.at[dstbs[b]],
                                  ssem.at[b]).wait()
        for k in range(_TAIL):
            j = _ITERS * _NB + k
            pltpu.sync_copy(dst_hbm.at[pl.ds(base + j * _CHUNK, _CHUNK)],
                            dstbs[k])
            pltpu.async_copy(
                h_hbm.at[src1d.at[pl.ds(j * _CHUNK, _CHUNK)]],
                rows_v.at[k], gsem.at[k]).wait()
            pltpu.async_copy(rows_v.at[k], acc_sh.at[dstbs[k]],
                             ssem.at[k], add=True)
        for k in range(_TAIL):
            pltpu.make_async_copy(rows_v.at[k], acc_sh.at[dstbs[k]],
                                  ssem.at[k]).wait()

        plsc.subcore_barrier()
        pltpu.sync_copy(acc_sh.at[pl.ds(r0, rows_per_sub)],
                        out_hbm.at[cid, pl.ds(r0, rows_per_sub)])

    return agg


_AGG = _agg_kernel()


def _mlp_layer(z2, w1, b1, w2, b2, gamma, beta):
    blk = 2048
    grid = _NP // blk

    def body(z_ref, w1_ref, b1_ref, w2_ref, b2_ref, g_ref, bt_ref, o_ref):
        z = z_ref[0] + z_ref[1]
        a = jnp.maximum(
            lax.dot(z, w1_ref[...], preferred_element_type=jnp.float32)
            + b1_ref[...], 0.0)
        zz = (lax.dot(a, w2_ref[...], preferred_element_type=jnp.float32)
              + b2_ref[...])
        scale = g_ref[...] * lax.rsqrt(jnp.float32(1.0 + _BN_EPS))
        res = jnp.maximum(zz * scale + bt_ref[...], 0.0)
        o_ref[...] = jnp.concatenate([res, jnp.zeros_like(res)], axis=1)

    return pl.pallas_call(
        body,
        grid=(grid,),
        in_specs=[
            pl.BlockSpec((_NC, blk, _D), lambda i: (0, i, 0)),
            pl.BlockSpec((_D, _HID), lambda i: (0, 0)),
            pl.BlockSpec((1, _HID), lambda i: (0, 0)),
            pl.BlockSpec((_HID, _HID), lambda i: (0, 0)),
            pl.BlockSpec((1, _HID), lambda i: (0, 0)),
            pl.BlockSpec((1, _HID), lambda i: (0, 0)),
            pl.BlockSpec((1, _HID), lambda i: (0, 0)),
        ],
        out_specs=pl.BlockSpec((blk, _D), lambda i: (i, 0)),
        out_shape=jax.ShapeDtypeStruct((_NP, _D), jnp.float32),
    )(z2, w1, b1.reshape(1, _HID), w2, b2.reshape(1, _HID),
      gamma.reshape(1, _HID), beta.reshape(1, _HID))


def _pool_head(h, batch_row, w1, b1, w2, b2):
    def body(h_ref, b_ref, w1_ref, b1_ref, w2_ref, b2_ref, o_ref):
        gids = lax.broadcasted_iota(jnp.int32, (_NG, _NP), 0)
        oh = (b_ref[...] == gids).astype(jnp.float32)
        sums = lax.dot(oh, h_ref[...], preferred_element_type=jnp.float32, precision=lax.Precision.HIGHEST)
        counts = jnp.sum(oh, axis=1, keepdims=True)
        pooled = sums / jnp.maximum(counts, 1.0)
        a = jnp.maximum(
            lax.dot(pooled, w1_ref[...], preferred_element_type=jnp.float32)
            + b1_ref[...], 0.0)
        o_ref[...] = (lax.dot(a, w2_ref[...],
                              preferred_element_type=jnp.float32)
                      + b2_ref[...])

    return pl.pallas_call(
        body,
        out_shape=jax.ShapeDtypeStruct((_NG, 1), jnp.float32),
    )(h, batch_row, w1, b1.reshape(1, -1), w2, b2.reshape(1, -1))


def _pad_rows(w, rows):
    return jnp.concatenate(
        [w, jnp.zeros((rows - w.shape[0], w.shape[1]), w.dtype)], axis=0)


def kernel(x, edge_index, edge_attr, batch, task_id, params):
    src = edge_index[0]
    dst = edge_index[1]
    pad = _NP - _N
    h = jnp.concatenate([x, jnp.zeros((pad, _F_IN), jnp.float32)], axis=0)
    zeros = jnp.zeros((_NP, _D), jnp.float32)
    for i in range(3):
        z2 = _AGG(h, src, dst, zeros)
        p = params['gin'][i]
        bn = params['bn'][i]
        h = _mlp_layer(z2, _pad_rows(p['w1'], _D), p['b1'], p['w2'], p['b2'],
                       bn['gamma'], bn['beta'])
    batch_row = jnp.concatenate(
        [batch, jnp.full((pad,), _NG, jnp.int32)]).reshape(1, _NP)
    hsel = jax.tree_util.tree_map(
        lambda a, b: jnp.where(task_id == 0, a, b),
        params['heads'][0], params['heads'][1],
    )
    return _pool_head(h, batch_row,
                      _pad_rows(hsel['w1'], _D), hsel['b1'],
                      hsel['w2'], hsel['b2'])


# speedup vs baseline: 1.1851x; 1.0112x over previous
import functools

import jax
import jax.numpy as jnp
from jax import lax
from jax.experimental import pallas as pl
from jax.experimental.pallas import tpu as pltpu
from jax.experimental.pallas import tpu_sc as plsc

_N = 10000
_NP = 10240
_E = 320000
_D = 128
_F_IN = 128
_HID = 64
_NG = 64
_BN_EPS = 1e-5

_NC = 2
_NS = 16
_NW = _NC * _NS
_CHUNK = 80
_NB = 3
_NCHUNK = _E // _NW // _CHUNK
_ITERS = _NCHUNK // _NB
_TAIL = _NCHUNK - _ITERS * _NB


def _agg_kernel():
    per_w = _E // _NW
    rows_per_sub = _NP // _NS
    mesh = plsc.VectorSubcoreMesh(core_axis_name="c", subcore_axis_name="s")

    @functools.partial(
        pl.kernel,
        mesh=mesh,
        out_type=jax.ShapeDtypeStruct((_NC, _NP, _D), jnp.float32),
        scratch_types=[
            pltpu.VMEM((per_w,), jnp.int32),
            pltpu.VMEM((_CHUNK,), jnp.int32),
            pltpu.VMEM((_CHUNK,), jnp.int32),
            pltpu.VMEM((_CHUNK,), jnp.int32),
            pltpu.VMEM((_NB, _CHUNK, _D), jnp.float32),
            pltpu.VMEM_SHARED((_NP, _D), jnp.float32),
            pltpu.SemaphoreType.DMA((_NB,)),
            pltpu.SemaphoreType.DMA((_NB,)),
            pltpu.SemaphoreType.DMA((_NB,)),
        ],
    )
    def agg(h_hbm, src_hbm, dst_hbm, zeros_hbm, out_hbm,
            src1d, db0, db1, db2, rows_v, acc_sh, gsem, ssem, isem):
        dstbs = [db0, db1, db2]
        cid = lax.axis_index("c")
        sid = lax.axis_index("s")
        wid = sid * _NC + cid
        r0 = sid * rows_per_sub
        base = wid * per_w

        pltpu.sync_copy(src_hbm.at[pl.ds(base, per_w)], src1d)

        @pl.when(cid == 0)
        def _():
            pltpu.sync_copy(h_hbm.at[pl.ds(r0, rows_per_sub)],
                            acc_sh.at[pl.ds(r0, rows_per_sub)])

        @pl.when(cid != 0)
        def _():
            pltpu.sync_copy(zeros_hbm.at[pl.ds(r0, rows_per_sub)],
                            acc_sh.at[pl.ds(r0, rows_per_sub)])

        plsc.subcore_barrier()

        def it(t, carry):
            ids, gds = [], []
            for b in range(_NB):
                j = t * _NB + b

                @pl.when(t > 0)
                def _(b=b):
                    pltpu.make_async_copy(
                        rows_v.at[b], acc_sh.at[dstbs[b]],
                        ssem.at[b]).wait()

                ids.append(pltpu.async_copy(
                    dst_hbm.at[pl.ds(base + j * _CHUNK, _CHUNK)],
                    dstbs[b], isem.at[b]))
                gds.append(pltpu.async_copy(
                    h_hbm.at[src1d.at[pl.ds(j * _CHUNK, _CHUNK)]],
                    rows_v.at[b], gsem.at[b]))
            for b in range(_NB):
                gds[b].wait()
                ids[b].wait()
                pltpu.async_copy(rows_v.at[b], acc_sh.at[dstbs[b]],
                                 ssem.at[b], add=True)
            return carry

        lax.fori_loop(0, _ITERS, it, 0)

        for b in range(_NB):
            pltpu.make_async_copy(rows_v.at[b], acc_sh.at[dstbs[b]],
                                  ssem.at[b]).wait()
        for k in range(_TAIL):
            j = _ITERS * _NB + k
            pltpu.sync_copy(dst_hbm.at[pl.ds(base + j * _CHUNK, _CHUNK)],
                            dstbs[k])
            pltpu.async_copy(
                h_hbm.at[src1d.at[pl.ds(j * _CHUNK, _CHUNK)]],
                rows_v.at[k], gsem.at[k]).wait()
            pltpu.async_copy(rows_v.at[k], acc_sh.at[dstbs[k]],
                             ssem.at[k], add=True)
        for k in range(_TAIL):
            pltpu.make_async_copy(rows_v.at[k], acc_sh.at[dstbs[k]],
                                  ssem.at[k]).wait()

        plsc.subcore_barrier()
        pltpu.sync_copy(acc_sh.at[pl.ds(r0, rows_per_sub)],
                        out_hbm.at[cid, pl.ds(r0, rows_per_sub)])

    return agg


@functools.lru_cache(maxsize=None)
def _agg_cached():
    return _agg_kernel()


def _AGG(h, src, dst, zeros):
    return _agg_cached()(h, src, dst, zeros)


def _mlp_layer(z2, w1, b1, w2, b2, gamma, beta):
    din = _D
    blk = 2048
    grid = _NP // blk

    def body(z_ref, w1_ref, b1_ref, w2_ref, b2_ref, g_ref, bt_ref, o_ref):
        z = z_ref[0] + z_ref[1]
        a = jnp.maximum(
            lax.dot(z, w1_ref[...], preferred_element_type=jnp.float32)
            + b1_ref[...], 0.0)
        zz = (lax.dot(a, w2_ref[...], preferred_element_type=jnp.float32)
              + b2_ref[...])
        scale = g_ref[...] * lax.rsqrt(jnp.float32(1.0 + _BN_EPS))
        res = jnp.maximum(zz * scale + bt_ref[...], 0.0)
        o_ref[...] = jnp.concatenate([res, jnp.zeros_like(res)], axis=1)

    return pl.pallas_call(
        body,
        grid=(grid,),
        in_specs=[
            pl.BlockSpec((_NC, blk, _D), lambda i: (0, i, 0)),
            pl.BlockSpec((_D, _HID), lambda i: (0, 0)),
            pl.BlockSpec((1, _HID), lambda i: (0, 0)),
            pl.BlockSpec((_HID, _HID), lambda i: (0, 0)),
            pl.BlockSpec((1, _HID), lambda i: (0, 0)),
            pl.BlockSpec((1, _HID), lambda i: (0, 0)),
            pl.BlockSpec((1, _HID), lambda i: (0, 0)),
        ],
        out_specs=pl.BlockSpec((blk, _D), lambda i: (i, 0)),
        out_shape=jax.ShapeDtypeStruct((_NP, _D), jnp.float32),
    )(z2, w1, b1.reshape(1, _HID), w2, b2.reshape(1, _HID),
      gamma.reshape(1, _HID), beta.reshape(1, _HID))


def _mlp3_pool_head(z2, w1, b1, w2, b2, gamma, beta, batch_row,
                    hw1, hb1, hw2, hb2):
    blk = 2048
    grid = _NP // blk

    def body(z_ref, w1_ref, b1_ref, w2_ref, b2_ref, g_ref, bt_ref,
             b_row_ref, hw1_ref, hb1_ref, hw2_ref, hb2_ref, o_ref,
             acc_ref, cnt_ref):
        i = pl.program_id(0)
        z = z_ref[0] + z_ref[1]
        a = jnp.maximum(
            lax.dot(z, w1_ref[...], preferred_element_type=jnp.float32)
            + b1_ref[...], 0.0)
        zz = (lax.dot(a, w2_ref[...], preferred_element_type=jnp.float32)
              + b2_ref[...])
        scale = g_ref[...] * lax.rsqrt(jnp.float32(1.0 + _BN_EPS))
        res = jnp.maximum(zz * scale + bt_ref[...], 0.0)
        gids = lax.broadcasted_iota(jnp.int32, (_NG, blk), 0)
        oh = (b_row_ref[...] == gids).astype(jnp.float32)
        part = lax.dot(oh, res, preferred_element_type=jnp.float32,
                       precision=lax.Precision.HIGHEST)
        cnt = jnp.sum(oh, axis=1, keepdims=True)

        @pl.when(i == 0)
        def _():
            acc_ref[...] = part
            cnt_ref[...] = cnt

        @pl.when(i > 0)
        def _():
            acc_ref[...] += part
            cnt_ref[...] += cnt

        @pl.when(i == grid - 1)
        def _():
            pooled = acc_ref[...] / jnp.maximum(cnt_ref[...], 1.0)
            ha = jnp.maximum(
                lax.dot(pooled, hw1_ref[...],
                        preferred_element_type=jnp.float32)
                + hb1_ref[...], 0.0)
            o_ref[...] = (lax.dot(ha, hw2_ref[...],
                                  preferred_element_type=jnp.float32)
                          + hb2_ref[...])

    return pl.pallas_call(
        body,
        grid=(grid,),
        in_specs=[
            pl.BlockSpec((_NC, blk, _D), lambda i: (0, i, 0)),
            pl.BlockSpec((_D, _HID), lambda i: (0, 0)),
            pl.BlockSpec((1, _HID), lambda i: (0, 0)),
            pl.BlockSpec((_HID, _HID), lambda i: (0, 0)),
            pl.BlockSpec((1, _HID), lambda i: (0, 0)),
            pl.BlockSpec((1, _HID), lambda i: (0, 0)),
            pl.BlockSpec((1, _HID), lambda i: (0, 0)),
            pl.BlockSpec((1, blk), lambda i: (0, i)),
            pl.BlockSpec((_HID, _HID // 2), lambda i: (0, 0)),
            pl.BlockSpec((1, _HID // 2), lambda i: (0, 0)),
            pl.BlockSpec((_HID // 2, 1), lambda i: (0, 0)),
            pl.BlockSpec((1, 1), lambda i: (0, 0)),
        ],
        out_specs=pl.BlockSpec((_NG, 1), lambda i: (0, 0)),
        out_shape=jax.ShapeDtypeStruct((_NG, 1), jnp.float32),
        scratch_shapes=[
            pltpu.VMEM((_NG, _HID), jnp.float32),
            pltpu.VMEM((_NG, 1), jnp.float32),
        ],
    )(z2, w1, b1.reshape(1, _HID), w2, b2.reshape(1, _HID),
      gamma.reshape(1, _HID), beta.reshape(1, _HID), batch_row,
      hw1, hb1.reshape(1, -1), hw2, hb2.reshape(1, -1))


def _pad_rows(w, rows):
    return jnp.concatenate(
        [w, jnp.zeros((rows - w.shape[0], w.shape[1]), w.dtype)], axis=0)


def kernel(x, edge_index, edge_attr, batch, task_id, params):
    src = edge_index[0]
    dst = edge_index[1]
    pad = _NP - _N
    h = jnp.concatenate([x, jnp.zeros((pad, _F_IN), jnp.float32)], axis=0)
    zeros = jnp.zeros((_NP, _D), jnp.float32)
    batch_row = jnp.concatenate(
        [batch, jnp.full((pad,), _NG, jnp.int32)]).reshape(1, _NP)
    hsel = jax.tree_util.tree_map(
        lambda a, b: jnp.where(task_id == 0, a, b),
        params['heads'][0], params['heads'][1],
    )
    for i in range(2):
        z2 = _AGG(h, src, dst, zeros)
        p = params['gin'][i]
        bn = params['bn'][i]
        h = _mlp_layer(z2, _pad_rows(p['w1'], _D), p['b1'], p['w2'],
                       p['b2'], bn['gamma'], bn['beta'])
    z2 = _AGG(h, src, dst, zeros)
    p = params['gin'][2]
    bn = params['bn'][2]
    return _mlp3_pool_head(z2, _pad_rows(p['w1'], _D), p['b1'], p['w2'],
                           p['b2'], bn['gamma'], bn['beta'], batch_row,
                           hsel['w1'], hsel['b1'], hsel['w2'], hsel['b2'])


# speedup vs baseline: 1.4240x; 1.2016x over previous
import functools

import jax
import jax.numpy as jnp
from jax import lax
from jax.experimental import pallas as pl
from jax.experimental.pallas import tpu as pltpu
from jax.experimental.pallas import tpu_sc as plsc

_N = 10000
_NP = 10240
_E = 320000
_D = 128
_F_IN = 128
_HID = 64
_NG = 64
_BN_EPS = 1e-5

_NC = 2
_NS = 16
_NW = _NC * _NS
_CHUNK = 80
_NB = 3
_NCHUNK = _E // _NW // _CHUNK
_ITERS = _NCHUNK // _NB
_TAIL = _NCHUNK - _ITERS * _NB


def _agg_kernel(d):
    per_w = _E // _NW
    rows_per_sub = _NP // _NS
    mesh = plsc.VectorSubcoreMesh(core_axis_name="c", subcore_axis_name="s")
    cp = (None if d == _D
          else pltpu.CompilerParams(use_tc_tiling_on_sc=False))

    @functools.partial(
        pl.kernel,
        mesh=mesh,
        out_type=jax.ShapeDtypeStruct((_NC, _NP, d), jnp.float32),
        compiler_params=cp,
        scratch_types=[
            pltpu.VMEM((per_w,), jnp.int32),
            pltpu.VMEM((_CHUNK,), jnp.int32),
            pltpu.VMEM((_CHUNK,), jnp.int32),
            pltpu.VMEM((_CHUNK,), jnp.int32),
            pltpu.VMEM((_NB, _CHUNK, d), jnp.float32),
            pltpu.VMEM_SHARED((_NP, d), jnp.float32),
            pltpu.SemaphoreType.DMA((_NB,)),
            pltpu.SemaphoreType.DMA((_NB,)),
            pltpu.SemaphoreType.DMA((_NB,)),
        ],
    )
    def agg(h_hbm, src_hbm, dst_hbm, zeros_hbm, out_hbm,
            src1d, db0, db1, db2, rows_v, acc_sh, gsem, ssem, isem):
        dstbs = [db0, db1, db2]
        cid = lax.axis_index("c")
        sid = lax.axis_index("s")
        wid = sid * _NC + cid
        r0 = sid * rows_per_sub
        base = wid * per_w

        pltpu.sync_copy(src_hbm.at[pl.ds(base, per_w)], src1d)

        @pl.when(cid == 0)
        def _():
            pltpu.sync_copy(h_hbm.at[pl.ds(r0, rows_per_sub)],
                            acc_sh.at[pl.ds(r0, rows_per_sub)])

        @pl.when(cid != 0)
        def _():
            pltpu.sync_copy(zeros_hbm.at[pl.ds(r0, rows_per_sub)],
                            acc_sh.at[pl.ds(r0, rows_per_sub)])

        plsc.subcore_barrier()

        def it(t, carry):
            ids, gds = [], []
            for b in range(_NB):
                j = t * _NB + b

                @pl.when(t > 0)
                def _(b=b):
                    pltpu.make_async_copy(
                        rows_v.at[b], acc_sh.at[dstbs[b]],
                        ssem.at[b]).wait()

                ids.append(pltpu.async_copy(
                    dst_hbm.at[pl.ds(base + j * _CHUNK, _CHUNK)],
                    dstbs[b], isem.at[b]))
                gds.append(pltpu.async_copy(
                    h_hbm.at[src1d.at[pl.ds(j * _CHUNK, _CHUNK)]],
                    rows_v.at[b], gsem.at[b]))
            for b in range(_NB):
                gds[b].wait()
                ids[b].wait()
                pltpu.async_copy(rows_v.at[b], acc_sh.at[dstbs[b]],
                                 ssem.at[b], add=True)
            return carry

        lax.fori_loop(0, _ITERS, it, 0)

        for b in range(_NB):
            pltpu.make_async_copy(rows_v.at[b], acc_sh.at[dstbs[b]],
                                  ssem.at[b]).wait()
        for k in range(_TAIL):
            j = _ITERS * _NB + k
            pltpu.sync_copy(dst_hbm.at[pl.ds(base + j * _CHUNK, _CHUNK)],
                            dstbs[k])
            pltpu.async_copy(
                h_hbm.at[src1d.at[pl.ds(j * _CHUNK, _CHUNK)]],
                rows_v.at[k], gsem.at[k]).wait()
            pltpu.async_copy(rows_v.at[k], acc_sh.at[dstbs[k]],
                             ssem.at[k], add=True)
        for k in range(_TAIL):
            pltpu.make_async_copy(rows_v.at[k], acc_sh.at[dstbs[k]],
                                  ssem.at[k]).wait()

        plsc.subcore_barrier()
        pltpu.sync_copy(acc_sh.at[pl.ds(r0, rows_per_sub)],
                        out_hbm.at[cid, pl.ds(r0, rows_per_sub)])

    return agg


@functools.lru_cache(maxsize=None)
def _agg_cached(d):
    return _agg_kernel(d)


def _AGG(h, src, dst, zeros):
    return _agg_cached(h.shape[1])(h, src, dst, zeros)


def _mlp_layer(z2, w1, b1, w2, b2, gamma, beta):
    din = z2.shape[2]
    blk = 2048
    grid = _NP // blk

    def body(z_ref, w1_ref, b1_ref, w2_ref, b2_ref, g_ref, bt_ref, o_ref):
        z = z_ref[0] + z_ref[1]
        a = jnp.maximum(
            lax.dot(z, w1_ref[...], preferred_element_type=jnp.float32)
            + b1_ref[...], 0.0)
        zz = (lax.dot(a, w2_ref[...], preferred_element_type=jnp.float32)
              + b2_ref[...])
        scale = g_ref[...] * lax.rsqrt(jnp.float32(1.0 + _BN_EPS))
        o_ref[...] = jnp.maximum(zz * scale + bt_ref[...], 0.0)

    return pl.pallas_call(
        body,
        grid=(grid,),
        in_specs=[
            pl.BlockSpec((_NC, blk, din), lambda i: (0, i, 0)),
            pl.BlockSpec((din, _HID), lambda i: (0, 0)),
            pl.BlockSpec((1, _HID), lambda i: (0, 0)),
            pl.BlockSpec((_HID, _HID), lambda i: (0, 0)),
            pl.BlockSpec((1, _HID), lambda i: (0, 0)),
            pl.BlockSpec((1, _HID), lambda i: (0, 0)),
            pl.BlockSpec((1, _HID), lambda i: (0, 0)),
        ],
        out_specs=pl.BlockSpec((blk, _HID), lambda i: (i, 0)),
        out_shape=jax.ShapeDtypeStruct((_NP, _HID), jnp.float32),
    )(z2, w1, b1.reshape(1, _HID), w2, b2.reshape(1, _HID),
      gamma.reshape(1, _HID), beta.reshape(1, _HID))


def _mlp3_pool_head(z2, w1, b1, w2, b2, gamma, beta, batch_row,
                    hw1, hb1, hw2, hb2):
    blk = 2048
    grid = _NP // blk

    def body(z_ref, w1_ref, b1_ref, w2_ref, b2_ref, g_ref, bt_ref,
             b_row_ref, hw1_ref, hb1_ref, hw2_ref, hb2_ref, o_ref,
             acc_ref, cnt_ref):
        i = pl.program_id(0)
        z = z_ref[0] + z_ref[1]
        a = jnp.maximum(
            lax.dot(z, w1_ref[...], preferred_element_type=jnp.float32)
            + b1_ref[...], 0.0)
        zz = (lax.dot(a, w2_ref[...], preferred_element_type=jnp.float32)
              + b2_ref[...])
        scale = g_ref[...] * lax.rsqrt(jnp.float32(1.0 + _BN_EPS))
        res = jnp.maximum(zz * scale + bt_ref[...], 0.0)
        gids = lax.broadcasted_iota(jnp.int32, (_NG, blk), 0)
        oh = (b_row_ref[...] == gids).astype(jnp.float32)
        part = lax.dot(oh, res, preferred_element_type=jnp.float32,
                       precision=lax.Precision.HIGHEST)
        cnt = jnp.sum(oh, axis=1, keepdims=True)

        @pl.when(i == 0)
        def _():
            acc_ref[...] = part
            cnt_ref[...] = cnt

        @pl.when(i > 0)
        def _():
            acc_ref[...] += part
            cnt_ref[...] += cnt

        @pl.when(i == grid - 1)
        def _():
            pooled = acc_ref[...] / jnp.maximum(cnt_ref[...], 1.0)
            ha = jnp.maximum(
                lax.dot(pooled, hw1_ref[...],
                        preferred_element_type=jnp.float32)
                + hb1_ref[...], 0.0)
            o_ref[...] = (lax.dot(ha, hw2_ref[...],
                                  preferred_element_type=jnp.float32)
                          + hb2_ref[...])

    return pl.pallas_call(
        body,
        grid=(grid,),
        in_specs=[
            pl.BlockSpec((_NC, blk, _HID), lambda i: (0, i, 0)),
            pl.BlockSpec((_HID, _HID), lambda i: (0, 0)),
            pl.BlockSpec((1, _HID), lambda i: (0, 0)),
            pl.BlockSpec((_HID, _HID), lambda i: (0, 0)),
            pl.BlockSpec((1, _HID), lambda i: (0, 0)),
            pl.BlockSpec((1, _HID), lambda i: (0, 0)),
            pl.BlockSpec((1, _HID), lambda i: (0, 0)),
            pl.BlockSpec((1, blk), lambda i: (0, i)),
            pl.BlockSpec((_HID, _HID // 2), lambda i: (0, 0)),
            pl.BlockSpec((1, _HID // 2), lambda i: (0, 0)),
            pl.BlockSpec((_HID // 2, 1), lambda i: (0, 0)),
            pl.BlockSpec((1, 1), lambda i: (0, 0)),
        ],
        out_specs=pl.BlockSpec((_NG, 1), lambda i: (0, 0)),
        out_shape=jax.ShapeDtypeStruct((_NG, 1), jnp.float32),
        scratch_shapes=[
            pltpu.VMEM((_NG, _HID), jnp.float32),
            pltpu.VMEM((_NG, 1), jnp.float32),
        ],
    )(z2, w1, b1.reshape(1, _HID), w2, b2.reshape(1, _HID),
      gamma.reshape(1, _HID), beta.reshape(1, _HID), batch_row,
      hw1, hb1.reshape(1, -1), hw2, hb2.reshape(1, -1))


def kernel(x, edge_index, edge_attr, batch, task_id, params):
    src = edge_index[0]
    dst = edge_index[1]
    pad = _NP - _N
    h = jnp.concatenate([x, jnp.zeros((pad, _F_IN), jnp.float32)], axis=0)
    batch_row = jnp.concatenate(
        [batch, jnp.full((pad,), _NG, jnp.int32)]).reshape(1, _NP)
    hsel = jax.tree_util.tree_map(
        lambda a, b: jnp.where(task_id == 0, a, b),
        params['heads'][0], params['heads'][1],
    )
    for i in range(2):
        z2 = _AGG(h, src, dst, jnp.zeros((_NP, h.shape[1]), jnp.float32))
        p = params['gin'][i]
        bn = params['bn'][i]
        h = _mlp_layer(z2, p['w1'], p['b1'], p['w2'],
                       p['b2'], bn['gamma'], bn['beta'])
    z2 = _AGG(h, src, dst, jnp.zeros((_NP, h.shape[1]), jnp.float32))
    p = params['gin'][2]
    bn = params['bn'][2]
    return _mlp3_pool_head(z2, p['w1'], p['b1'], p['w2'],
                           p['b2'], bn['gamma'], bn['beta'], batch_row,
                           hsel['w1'], hsel['b1'], hsel['w2'], hsel['b2'])


# speedup vs baseline: 1.5010x; 1.0541x over previous
import functools

import jax
import jax.numpy as jnp
from jax import lax
from jax.experimental import pallas as pl
from jax.experimental.pallas import tpu as pltpu
from jax.experimental.pallas import tpu_sc as plsc

_N = 10000
_NP = 10240
_E = 320000
_D = 128
_F_IN = 128
_HID = 64
_NG = 64
_BN_EPS = 1e-5

_NC = 2
_NS = 16
_NW = _NC * _NS
_CHUNK = 80
_NCHUNK = _E // _NW // _CHUNK


def _agg_kernel(d):
    per_w = _E // _NW
    rows_per_sub = _NP // _NS
    mesh = plsc.VectorSubcoreMesh(core_axis_name="c", subcore_axis_name="s")
    cp = (None if d == _D
          else pltpu.CompilerParams(use_tc_tiling_on_sc=False))
    _NB = 3 if d == _D else 6
    _ITERS = _NCHUNK // _NB
    _TAIL = _NCHUNK - _ITERS * _NB

    @functools.partial(
        pl.kernel,
        mesh=mesh,
        out_type=jax.ShapeDtypeStruct((_NC, _NP, d), jnp.float32),
        compiler_params=cp,
        scratch_types=(
            [pltpu.VMEM((per_w,), jnp.int32)]
            + [pltpu.VMEM((_CHUNK,), jnp.int32) for _ in range(_NB)]
            + [
                pltpu.VMEM((_NB, _CHUNK, d), jnp.float32),
                pltpu.VMEM_SHARED((_NP, d), jnp.float32),
                pltpu.SemaphoreType.DMA((_NB,)),
                pltpu.SemaphoreType.DMA((_NB,)),
                pltpu.SemaphoreType.DMA((_NB,)),
            ]
        ),
    )
    def agg(h_hbm, src_hbm, dst_hbm, zeros_hbm, out_hbm, *scr):
        src1d = scr[0]
        dstbs = list(scr[1:1 + _NB])
        rows_v, acc_sh, gsem, ssem, isem = scr[1 + _NB:]
        cid = lax.axis_index("c")
        sid = lax.axis_index("s")
        wid = sid * _NC + cid
        r0 = sid * rows_per_sub
        base = wid * per_w

        pltpu.sync_copy(src_hbm.at[pl.ds(base, per_w)], src1d)

        @pl.when(cid == 0)
        def _():
            pltpu.sync_copy(h_hbm.at[pl.ds(r0, rows_per_sub)],
                            acc_sh.at[pl.ds(r0, rows_per_sub)])

        @pl.when(cid != 0)
        def _():
            pltpu.sync_copy(zeros_hbm.at[pl.ds(r0, rows_per_sub)],
                            acc_sh.at[pl.ds(r0, rows_per_sub)])

        plsc.subcore_barrier()

        def it(t, carry):
            ids, gds = [], []
            for b in range(_NB):
                j = t * _NB + b

                @pl.when(t > 0)
                def _(b=b):
                    pltpu.make_async_copy(
                        rows_v.at[b], acc_sh.at[dstbs[b]],
                        ssem.at[b]).wait()

                ids.append(pltpu.async_copy(
                    dst_hbm.at[pl.ds(base + j * _CHUNK, _CHUNK)],
                    dstbs[b], isem.at[b]))
                gds.append(pltpu.async_copy(
                    h_hbm.at[src1d.at[pl.ds(j * _CHUNK, _CHUNK)]],
                    rows_v.at[b], gsem.at[b]))
            for b in range(_NB):
                gds[b].wait()
                ids[b].wait()
                pltpu.async_copy(rows_v.at[b], acc_sh.at[dstbs[b]],
                                 ssem.at[b], add=True)
            return carry

        lax.fori_loop(0, _ITERS, it, 0)

        for b in range(_NB):
            pltpu.make_async_copy(rows_v.at[b], acc_sh.at[dstbs[b]],
                                  ssem.at[b]).wait()
        for k in range(_TAIL):
            j = _ITERS * _NB + k
            pltpu.sync_copy(dst_hbm.at[pl.ds(base + j * _CHUNK, _CHUNK)],
                            dstbs[k])
            pltpu.async_copy(
                h_hbm.at[src1d.at[pl.ds(j * _CHUNK, _CHUNK)]],
                rows_v.at[k], gsem.at[k]).wait()
            pltpu.async_copy(rows_v.at[k], acc_sh.at[dstbs[k]],
                             ssem.at[k], add=True)
        for k in range(_TAIL):
            pltpu.make_async_copy(rows_v.at[k], acc_sh.at[dstbs[k]],
                                  ssem.at[k]).wait()

        plsc.subcore_barrier()
        pltpu.sync_copy(acc_sh.at[pl.ds(r0, rows_per_sub)],
                        out_hbm.at[cid, pl.ds(r0, rows_per_sub)])

    return agg


@functools.lru_cache(maxsize=None)
def _agg_cached(d):
    return _agg_kernel(d)


def _AGG(h, src, dst, zeros):
    return _agg_cached(h.shape[1])(h, src, dst, zeros)


def _mlp_layer(z2, w1, b1, w2, b2, gamma, beta):
    din = z2.shape[2]
    blk = 2048
    grid = _NP // blk

    def body(z_ref, w1_ref, b1_ref, w2_ref, b2_ref, g_ref, bt_ref, o_ref):
        z = z_ref[0] + z_ref[1]
        a = jnp.maximum(
            lax.dot(z, w1_ref[...], preferred_element_type=jnp.float32)
            + b1_ref[...], 0.0)
        zz = (lax.dot(a, w2_ref[...], preferred_element_type=jnp.float32)
              + b2_ref[...])
        scale = g_ref[...] * lax.rsqrt(jnp.float32(1.0 + _BN_EPS))
        o_ref[...] = jnp.maximum(zz * scale + bt_ref[...], 0.0)

    return pl.pallas_call(
        body,
        grid=(grid,),
        in_specs=[
            pl.BlockSpec((_NC, blk, din), lambda i: (0, i, 0)),
            pl.BlockSpec((din, _HID), lambda i: (0, 0)),
            pl.BlockSpec((1, _HID), lambda i: (0, 0)),
            pl.BlockSpec((_HID, _HID), lambda i: (0, 0)),
            pl.BlockSpec((1, _HID), lambda i: (0, 0)),
            pl.BlockSpec((1, _HID), lambda i: (0, 0)),
            pl.BlockSpec((1, _HID), lambda i: (0, 0)),
        ],
        out_specs=pl.BlockSpec((blk, _HID), lambda i: (i, 0)),
        out_shape=jax.ShapeDtypeStruct((_NP, _HID), jnp.float32),
    )(z2, w1, b1.reshape(1, _HID), w2, b2.reshape(1, _HID),
      gamma.reshape(1, _HID), beta.reshape(1, _HID))


def _mlp3_pool_head(z2, w1, b1, w2, b2, gamma, beta, batch_row,
                    hw1, hb1, hw2, hb2):
    blk = 2048
    grid = _NP // blk

    def body(z_ref, w1_ref, b1_ref, w2_ref, b2_ref, g_ref, bt_ref,
             b_row_ref, hw1_ref, hb1_ref, hw2_ref, hb2_ref, o_ref,
             acc_ref, cnt_ref):
        i = pl.program_id(0)
        z = z_ref[0] + z_ref[1]
        a = jnp.maximum(
            lax.dot(z, w1_ref[...], preferred_element_type=jnp.float32)
            + b1_ref[...], 0.0)
        zz = (lax.dot(a, w2_ref[...], preferred_element_type=jnp.float32)
              + b2_ref[...])
        scale = g_ref[...] * lax.rsqrt(jnp.float32(1.0 + _BN_EPS))
        res = jnp.maximum(zz * scale + bt_ref[...], 0.0)
        gids = lax.broadcasted_iota(jnp.int32, (_NG, blk), 0)
        oh = (b_row_ref[...] == gids).astype(jnp.float32)
        part = lax.dot(oh, res, preferred_element_type=jnp.float32,
                       precision=lax.Precision.HIGHEST)
        cnt = jnp.sum(oh, axis=1, keepdims=True)

        @pl.when(i == 0)
        def _():
            acc_ref[...] = part
            cnt_ref[...] = cnt

        @pl.when(i > 0)
        def _():
            acc_ref[...] += part
            cnt_ref[...] += cnt

        @pl.when(i == grid - 1)
        def _():
            pooled = acc_ref[...] / jnp.maximum(cnt_ref[...], 1.0)
            ha = jnp.maximum(
                lax.dot(pooled, hw1_ref[...],
                        preferred_element_type=jnp.float32)
                + hb1_ref[...], 0.0)
            o_ref[...] = (lax.dot(ha, hw2_ref[...],
                                  preferred_element_type=jnp.float32)
                          + hb2_ref[...])

    return pl.pallas_call(
        body,
        grid=(grid,),
        in_specs=[
            pl.BlockSpec((_NC, blk, _HID), lambda i: (0, i, 0)),
            pl.BlockSpec((_HID, _HID), lambda i: (0, 0)),
            pl.BlockSpec((1, _HID), lambda i: (0, 0)),
            pl.BlockSpec((_HID, _HID), lambda i: (0, 0)),
            pl.BlockSpec((1, _HID), lambda i: (0, 0)),
            pl.BlockSpec((1, _HID), lambda i: (0, 0)),
            pl.BlockSpec((1, _HID), lambda i: (0, 0)),
            pl.BlockSpec((1, blk), lambda i: (0, i)),
            pl.BlockSpec((_HID, _HID // 2), lambda i: (0, 0)),
            pl.BlockSpec((1, _HID // 2), lambda i: (0, 0)),
            pl.BlockSpec((_HID // 2, 1), lambda i: (0, 0)),
            pl.BlockSpec((1, 1), lambda i: (0, 0)),
        ],
        out_specs=pl.BlockSpec((_NG, 1), lambda i: (0, 0)),
        out_shape=jax.ShapeDtypeStruct((_NG, 1), jnp.float32),
        scratch_shapes=[
            pltpu.VMEM((_NG, _HID), jnp.float32),
            pltpu.VMEM((_NG, 1), jnp.float32),
        ],
    )(z2, w1, b1.reshape(1, _HID), w2, b2.reshape(1, _HID),
      gamma.reshape(1, _HID), beta.reshape(1, _HID), batch_row,
      hw1, hb1.reshape(1, -1), hw2, hb2.reshape(1, -1))


def kernel(x, edge_index, edge_attr, batch, task_id, params):
    src = edge_index[0]
    dst = edge_index[1]
    pad = _NP - _N
    h = jnp.concatenate([x, jnp.zeros((pad, _F_IN), jnp.float32)], axis=0)
    batch_row = jnp.concatenate(
        [batch, jnp.full((pad,), _NG, jnp.int32)]).reshape(1, _NP)
    hsel = jax.tree_util.tree_map(
        lambda a, b: jnp.where(task_id == 0, a, b),
        params['heads'][0], params['heads'][1],
    )
    for i in range(2):
        z2 = _AGG(h, src, dst, jnp.zeros((_NP, h.shape[1]), jnp.float32))
        p = params['gin'][i]
        bn = params['bn'][i]
        h = _mlp_layer(z2, p['w1'], p['b1'], p['w2'],
                       p['b2'], bn['gamma'], bn['beta'])
    z2 = _AGG(h, src, dst, jnp.zeros((_NP, h.shape[1]), jnp.float32))
    p = params['gin'][2]
    bn = params['bn'][2]
    return _mlp3_pool_head(z2, p['w1'], p['b1'], p['w2'],
                           p['b2'], bn['gamma'], bn['beta'], batch_row,
                           hsel['w1'], hsel['b1'], hsel['w2'], hsel['b2'])


# speedup vs baseline: 1.5011x; 1.0001x over previous
import functools

import jax
import jax.numpy as jnp
from jax import lax
from jax.experimental import pallas as pl
from jax.experimental.pallas import tpu as pltpu
from jax.experimental.pallas import tpu_sc as plsc

_N = 10000
_NP = 10240
_E = 320000
_D = 128
_F_IN = 128
_HID = 64
_NG = 64
_BN_EPS = 1e-5

_NC = 2
_NS = 16
_NW = _NC * _NS
_CHUNK = 80
_NCHUNK = _E // _NW // _CHUNK


def _agg_kernel(d):
    per_w = _E // _NW
    rows_per_sub = _NP // _NS
    mesh = plsc.VectorSubcoreMesh(core_axis_name="c", subcore_axis_name="s")
    cp = pltpu.CompilerParams(use_tc_tiling_on_sc=False)
    _NB = 3 if d == _D else 6
    _ITERS = _NCHUNK // _NB
    _TAIL = _NCHUNK - _ITERS * _NB

    @functools.partial(
        pl.kernel,
        mesh=mesh,
        out_type=jax.ShapeDtypeStruct((_NC, _NP, d), jnp.float32),
        compiler_params=cp,
        scratch_types=(
            [pltpu.VMEM((per_w,), jnp.int32)]
            + [pltpu.VMEM((_CHUNK,), jnp.int32) for _ in range(_NB)]
            + [
                pltpu.VMEM((_NB, _CHUNK, d), jnp.float32),
                pltpu.VMEM_SHARED((_NP, d), jnp.float32),
                pltpu.SemaphoreType.DMA((_NB,)),
                pltpu.SemaphoreType.DMA((_NB,)),
                pltpu.SemaphoreType.DMA((_NB,)),
            ]
        ),
    )
    def agg(h_hbm, src_hbm, dst_hbm, zeros_hbm, out_hbm, *scr):
        src1d = scr[0]
        dstbs = list(scr[1:1 + _NB])
        rows_v, acc_sh, gsem, ssem, isem = scr[1 + _NB:]
        cid = lax.axis_index("c")
        sid = lax.axis_index("s")
        wid = sid * _NC + cid
        r0 = sid * rows_per_sub
        base = wid * per_w

        pltpu.sync_copy(src_hbm.at[pl.ds(base, per_w)], src1d)

        @pl.when(cid == 0)
        def _():
            pltpu.sync_copy(h_hbm.at[pl.ds(r0, rows_per_sub)],
                            acc_sh.at[pl.ds(r0, rows_per_sub)])

        @pl.when(cid != 0)
        def _():
            pltpu.sync_copy(zeros_hbm.at[pl.ds(r0, rows_per_sub)],
                            acc_sh.at[pl.ds(r0, rows_per_sub)])

        plsc.subcore_barrier()

        def it(t, carry):
            ids, gds = [], []
            for b in range(_NB):
                j = t * _NB + b

                @pl.when(t > 0)
                def _(b=b):
                    pltpu.make_async_copy(
                        rows_v.at[b], acc_sh.at[dstbs[b]],
                        ssem.at[b]).wait()

                ids.append(pltpu.async_copy(
                    dst_hbm.at[pl.ds(base + j * _CHUNK, _CHUNK)],
                    dstbs[b], isem.at[b]))
                gds.append(pltpu.async_copy(
                    h_hbm.at[src1d.at[pl.ds(j * _CHUNK, _CHUNK)]],
                    rows_v.at[b], gsem.at[b]))
            for b in range(_NB):
                gds[b].wait()
                ids[b].wait()
                pltpu.async_copy(rows_v.at[b], acc_sh.at[dstbs[b]],
                                 ssem.at[b], add=True)
            return carry

        lax.fori_loop(0, _ITERS, it, 0)

        for b in range(_NB):
            pltpu.make_async_copy(rows_v.at[b], acc_sh.at[dstbs[b]],
                                  ssem.at[b]).wait()
        for k in range(_TAIL):
            j = _ITERS * _NB + k
            pltpu.sync_copy(dst_hbm.at[pl.ds(base + j * _CHUNK, _CHUNK)],
                            dstbs[k])
            pltpu.async_copy(
                h_hbm.at[src1d.at[pl.ds(j * _CHUNK, _CHUNK)]],
                rows_v.at[k], gsem.at[k]).wait()
            pltpu.async_copy(rows_v.at[k], acc_sh.at[dstbs[k]],
                             ssem.at[k], add=True)
        for k in range(_TAIL):
            pltpu.make_async_copy(rows_v.at[k], acc_sh.at[dstbs[k]],
                                  ssem.at[k]).wait()

        plsc.subcore_barrier()
        pltpu.sync_copy(acc_sh.at[pl.ds(r0, rows_per_sub)],
                        out_hbm.at[cid, pl.ds(r0, rows_per_sub)])

    return agg


@functools.lru_cache(maxsize=None)
def _agg_cached(d):
    return _agg_kernel(d)


def _AGG(h, src, dst, zeros):
    return _agg_cached(h.shape[1])(h, src, dst, zeros)


def _mlp_layer(z2, w1, b1, w2, b2, gamma, beta):
    din = z2.shape[2]
    blk = 2048
    grid = _NP // blk

    def body(z_ref, w1_ref, b1_ref, w2_ref, b2_ref, g_ref, bt_ref, o_ref):
        z = z_ref[0] + z_ref[1]
        a = jnp.maximum(
            lax.dot(z, w1_ref[...], preferred_element_type=jnp.float32)
            + b1_ref[...], 0.0)
        zz = (lax.dot(a, w2_ref[...], preferred_element_type=jnp.float32)
              + b2_ref[...])
        scale = g_ref[...] * lax.rsqrt(jnp.float32(1.0 + _BN_EPS))
        o_ref[...] = jnp.maximum(zz * scale + bt_ref[...], 0.0)

    return pl.pallas_call(
        body,
        grid=(grid,),
        in_specs=[
            pl.BlockSpec((_NC, blk, din), lambda i: (0, i, 0)),
            pl.BlockSpec((din, _HID), lambda i: (0, 0)),
            pl.BlockSpec((1, _HID), lambda i: (0, 0)),
            pl.BlockSpec((_HID, _HID), lambda i: (0, 0)),
            pl.BlockSpec((1, _HID), lambda i: (0, 0)),
            pl.BlockSpec((1, _HID), lambda i: (0, 0)),
            pl.BlockSpec((1, _HID), lambda i: (0, 0)),
        ],
        out_specs=pl.BlockSpec((blk, _HID), lambda i: (i, 0)),
        out_shape=jax.ShapeDtypeStruct((_NP, _HID), jnp.float32),
    )(z2, w1, b1.reshape(1, _HID), w2, b2.reshape(1, _HID),
      gamma.reshape(1, _HID), beta.reshape(1, _HID))


def _mlp3_pool_head(z2, w1, b1, w2, b2, gamma, beta, batch_row,
                    hw1, hb1, hw2, hb2):
    blk = 2048
    grid = _NP // blk

    def body(z_ref, w1_ref, b1_ref, w2_ref, b2_ref, g_ref, bt_ref,
             b_row_ref, hw1_ref, hb1_ref, hw2_ref, hb2_ref, o_ref,
             acc_ref, cnt_ref):
        i = pl.program_id(0)
        z = z_ref[0] + z_ref[1]
        a = jnp.maximum(
            lax.dot(z, w1_ref[...], preferred_element_type=jnp.float32)
            + b1_ref[...], 0.0)
        zz = (lax.dot(a, w2_ref[...], preferred_element_type=jnp.float32)
              + b2_ref[...])
        scale = g_ref[...] * lax.rsqrt(jnp.float32(1.0 + _BN_EPS))
        res = jnp.maximum(zz * scale + bt_ref[...], 0.0)
        gids = lax.broadcasted_iota(jnp.int32, (_NG, blk), 0)
        oh = (b_row_ref[...] == gids).astype(jnp.float32)
        part = lax.dot(oh, res, preferred_element_type=jnp.float32,
                       precision=lax.Precision.HIGHEST)
        cnt = jnp.sum(oh, axis=1, keepdims=True)

        @pl.when(i == 0)
        def _():
            acc_ref[...] = part
            cnt_ref[...] = cnt

        @pl.when(i > 0)
        def _():
            acc_ref[...] += part
            cnt_ref[...] += cnt

        @pl.when(i == grid - 1)
        def _():
            pooled = acc_ref[...] / jnp.maximum(cnt_ref[...], 1.0)
            ha = jnp.maximum(
                lax.dot(pooled, hw1_ref[...],
                        preferred_element_type=jnp.float32)
                + hb1_ref[...], 0.0)
            o_ref[...] = (lax.dot(ha, hw2_ref[...],
                                  preferred_element_type=jnp.float32)
                          + hb2_ref[...])

    return pl.pallas_call(
        body,
        grid=(grid,),
        in_specs=[
            pl.BlockSpec((_NC, blk, _HID), lambda i: (0, i, 0)),
            pl.BlockSpec((_HID, _HID), lambda i: (0, 0)),
            pl.BlockSpec((1, _HID), lambda i: (0, 0)),
            pl.BlockSpec((_HID, _HID), lambda i: (0, 0)),
            pl.BlockSpec((1, _HID), lambda i: (0, 0)),
            pl.BlockSpec((1, _HID), lambda i: (0, 0)),
            pl.BlockSpec((1, _HID), lambda i: (0, 0)),
            pl.BlockSpec((1, blk), lambda i: (0, i)),
            pl.BlockSpec((_HID, _HID // 2), lambda i: (0, 0)),
            pl.BlockSpec((1, _HID // 2), lambda i: (0, 0)),
            pl.BlockSpec((_HID // 2, 1), lambda i: (0, 0)),
            pl.BlockSpec((1, 1), lambda i: (0, 0)),
        ],
        out_specs=pl.BlockSpec((_NG, 1), lambda i: (0, 0)),
        out_shape=jax.ShapeDtypeStruct((_NG, 1), jnp.float32),
        scratch_shapes=[
            pltpu.VMEM((_NG, _HID), jnp.float32),
            pltpu.VMEM((_NG, 1), jnp.float32),
        ],
    )(z2, w1, b1.reshape(1, _HID), w2, b2.reshape(1, _HID),
      gamma.reshape(1, _HID), beta.reshape(1, _HID), batch_row,
      hw1, hb1.reshape(1, -1), hw2, hb2.reshape(1, -1))


def kernel(x, edge_index, edge_attr, batch, task_id, params):
    src = edge_index[0]
    dst = edge_index[1]
    pad = _NP - _N
    h = jnp.concatenate([x, jnp.zeros((pad, _F_IN), jnp.float32)], axis=0)
    batch_row = jnp.concatenate(
        [batch, jnp.full((pad,), _NG, jnp.int32)]).reshape(1, _NP)
    hsel = jax.tree_util.tree_map(
        lambda a, b: jnp.where(task_id == 0, a, b),
        params['heads'][0], params['heads'][1],
    )
    for i in range(2):
        z2 = _AGG(h, src, dst, jnp.zeros((_NP, h.shape[1]), jnp.float32))
        p = params['gin'][i]
        bn = params['bn'][i]
        h = _mlp_layer(z2, p['w1'], p['b1'], p['w2'],
                       p['b2'], bn['gamma'], bn['beta'])
    z2 = _AGG(h, src, dst, jnp.zeros((_NP, h.shape[1]), jnp.float32))
    p = params['gin'][2]
    bn = params['bn'][2]
    return _mlp3_pool_head(z2, p['w1'], p['b1'], p['w2'],
                           p['b2'], bn['gamma'], bn['beta'], batch_row,
                           hsel['w1'], hsel['b1'], hsel['w2'], hsel['b2'])


# speedup vs baseline: 1.5038x; 1.0018x over previous
import functools

import jax
import jax.numpy as jnp
from jax import lax
from jax.experimental import pallas as pl
from jax.experimental.pallas import tpu as pltpu
from jax.experimental.pallas import tpu_sc as plsc

_N = 10000
_NP = 10240
_E = 320000
_D = 128
_F_IN = 128
_HID = 64
_NG = 64
_BN_EPS = 1e-5

_NC = 2
_NS = 16
_NW = _NC * _NS
_CHUNK = 80
_NCHUNK = _E // _NW // _CHUNK


def _agg_kernel(d):
    per_w = _E // _NW
    rows_per_sub = _NP // _NS
    mesh = plsc.VectorSubcoreMesh(core_axis_name="c", subcore_axis_name="s")
    cp = (None if d == _D
          else pltpu.CompilerParams(use_tc_tiling_on_sc=False))
    _NB = 3 if d == _D else 6
    _ITERS = _NCHUNK // _NB
    _TAIL = _NCHUNK - _ITERS * _NB

    @functools.partial(
        pl.kernel,
        mesh=mesh,
        out_type=jax.ShapeDtypeStruct((_NC, _NP, d), jnp.float32),
        compiler_params=cp,
        scratch_types=(
            [pltpu.VMEM((per_w,), jnp.int32)]
            + [pltpu.VMEM((_CHUNK,), jnp.int32) for _ in range(_NB)]
            + [
                pltpu.VMEM((_NB, _CHUNK, d), jnp.float32),
                pltpu.VMEM_SHARED((_NP, d), jnp.float32),
                pltpu.SemaphoreType.DMA((_NB,)),
                pltpu.SemaphoreType.DMA((_NB,)),
                pltpu.SemaphoreType.DMA((_NB,)),
            ]
        ),
    )
    def agg(h_hbm, src_hbm, dst_hbm, zeros_hbm, out_hbm, *scr):
        src1d = scr[0]
        dstbs = list(scr[1:1 + _NB])
        rows_v, acc_sh, gsem, ssem, isem = scr[1 + _NB:]
        cid = lax.axis_index("c")
        sid = lax.axis_index("s")
        wid = sid * _NC + cid
        r0 = sid * rows_per_sub
        base = wid * per_w

        pltpu.sync_copy(src_hbm.at[pl.ds(base, per_w)], src1d)

        @pl.when(cid == 0)
        def _():
            pltpu.sync_copy(h_hbm.at[pl.ds(r0, rows_per_sub)],
                            acc_sh.at[pl.ds(r0, rows_per_sub)])

        @pl.when(cid != 0)
        def _():
            pltpu.sync_copy(zeros_hbm.at[pl.ds(r0, rows_per_sub)],
                            acc_sh.at[pl.ds(r0, rows_per_sub)])

        plsc.subcore_barrier()

        def it(t, carry):
            ids, gds = [], []
            for b in range(_NB):
                j = t * _NB + b

                @pl.when(t > 0)
                def _(b=b):
                    pltpu.make_async_copy(
                        rows_v.at[b], acc_sh.at[dstbs[b]],
                        ssem.at[b]).wait()

                ids.append(pltpu.async_copy(
                    dst_hbm.at[pl.ds(base + j * _CHUNK, _CHUNK)],
                    dstbs[b], isem.at[b]))
                gds.append(pltpu.async_copy(
                    h_hbm.at[src1d.at[pl.ds(j * _CHUNK, _CHUNK)]],
                    rows_v.at[b], gsem.at[b]))
            for b in range(_NB):
                gds[b].wait()
                ids[b].wait()
                pltpu.async_copy(rows_v.at[b], acc_sh.at[dstbs[b]],
                                 ssem.at[b], add=True)
            return carry

        lax.fori_loop(0, _ITERS, it, 0)

        for b in range(_NB):
            pltpu.make_async_copy(rows_v.at[b], acc_sh.at[dstbs[b]],
                                  ssem.at[b]).wait()
        for k in range(_TAIL):
            j = _ITERS * _NB + k
            pltpu.sync_copy(dst_hbm.at[pl.ds(base + j * _CHUNK, _CHUNK)],
                            dstbs[k])
            pltpu.async_copy(
                h_hbm.at[src1d.at[pl.ds(j * _CHUNK, _CHUNK)]],
                rows_v.at[k], gsem.at[k]).wait()
            pltpu.async_copy(rows_v.at[k], acc_sh.at[dstbs[k]],
                             ssem.at[k], add=True)
        for k in range(_TAIL):
            pltpu.make_async_copy(rows_v.at[k], acc_sh.at[dstbs[k]],
                                  ssem.at[k]).wait()

        plsc.subcore_barrier()
        pltpu.sync_copy(acc_sh.at[pl.ds(r0, rows_per_sub)],
                        out_hbm.at[cid, pl.ds(r0, rows_per_sub)])

    return agg


@functools.lru_cache(maxsize=None)
def _agg_cached(d):
    return _agg_kernel(d)


def _AGG(h, src, dst, zeros):
    return _agg_cached(h.shape[1])(h, src, dst, zeros)


def _mlp_layer(z2, w1, b1, w2, b2, gamma, beta):
    din = z2.shape[2]
    blk = 2048
    grid = _NP // blk

    def body(z_ref, w1_ref, b1_ref, w2_ref, b2_ref, g_ref, bt_ref, o_ref):
        z = z_ref[0] + z_ref[1]
        a = jnp.maximum(
            lax.dot(z, w1_ref[...], preferred_element_type=jnp.float32)
            + b1_ref[...], 0.0)
        zz = (lax.dot(a, w2_ref[...], preferred_element_type=jnp.float32)
              + b2_ref[...])
        scale = g_ref[...] * lax.rsqrt(jnp.float32(1.0 + _BN_EPS))
        o_ref[...] = jnp.maximum(zz * scale + bt_ref[...], 0.0)

    return pl.pallas_call(
        body,
        grid=(grid,),
        in_specs=[
            pl.BlockSpec((_NC, blk, din), lambda i: (0, i, 0)),
            pl.BlockSpec((din, _HID), lambda i: (0, 0)),
            pl.BlockSpec((1, _HID), lambda i: (0, 0)),
            pl.BlockSpec((_HID, _HID), lambda i: (0, 0)),
            pl.BlockSpec((1, _HID), lambda i: (0, 0)),
            pl.BlockSpec((1, _HID), lambda i: (0, 0)),
            pl.BlockSpec((1, _HID), lambda i: (0, 0)),
        ],
        out_specs=pl.BlockSpec((blk, _HID), lambda i: (i, 0)),
        out_shape=jax.ShapeDtypeStruct((_NP, _HID), jnp.float32),
    )(z2, w1, b1.reshape(1, _HID), w2, b2.reshape(1, _HID),
      gamma.reshape(1, _HID), beta.reshape(1, _HID))


def _mlp3_pool_head(z2, w1, b1, w2, b2, gamma, beta, batch_row,
                    hw1, hb1, hw2, hb2):
    blk = 2048
    grid = _NP // blk

    def body(z_ref, w1_ref, b1_ref, w2_ref, b2_ref, g_ref, bt_ref,
             b_row_ref, hw1_ref, hb1_ref, hw2_ref, hb2_ref, o_ref,
             acc_ref, cnt_ref):
        i = pl.program_id(0)
        z = z_ref[0] + z_ref[1]
        a = jnp.maximum(
            lax.dot(z, w1_ref[...], preferred_element_type=jnp.float32)
            + b1_ref[...], 0.0)
        zz = (lax.dot(a, w2_ref[...], preferred_element_type=jnp.float32)
              + b2_ref[...])
        scale = g_ref[...] * lax.rsqrt(jnp.float32(1.0 + _BN_EPS))
        res = jnp.maximum(zz * scale + bt_ref[...], 0.0)
        gids = lax.broadcasted_iota(jnp.int32, (_NG, blk), 0)
        oh = (b_row_ref[...] == gids).astype(jnp.float32)
        part = lax.dot(oh, res, preferred_element_type=jnp.float32,
                       precision=lax.Precision.HIGHEST)
        cnt = jnp.sum(oh, axis=1, keepdims=True)

        @pl.when(i == 0)
        def _():
            acc_ref[...] = part
            cnt_ref[...] = cnt

        @pl.when(i > 0)
        def _():
            acc_ref[...] += part
            cnt_ref[...] += cnt

        @pl.when(i == grid - 1)
        def _():
            pooled = acc_ref[...] / jnp.maximum(cnt_ref[...], 1.0)
            ha = jnp.maximum(
                lax.dot(pooled, hw1_ref[...],
                        preferred_element_type=jnp.float32)
                + hb1_ref[...], 0.0)
            o_ref[...] = (lax.dot(ha, hw2_ref[...],
                                  preferred_element_type=jnp.float32)
                          + hb2_ref[...])

    return pl.pallas_call(
        body,
        grid=(grid,),
        in_specs=[
            pl.BlockSpec((_NC, blk, _HID), lambda i: (0, i, 0)),
            pl.BlockSpec((_HID, _HID), lambda i: (0, 0)),
            pl.BlockSpec((1, _HID), lambda i: (0, 0)),
            pl.BlockSpec((_HID, _HID), lambda i: (0, 0)),
            pl.BlockSpec((1, _HID), lambda i: (0, 0)),
            pl.BlockSpec((1, _HID), lambda i: (0, 0)),
            pl.BlockSpec((1, _HID), lambda i: (0, 0)),
            pl.BlockSpec((1, blk), lambda i: (0, i)),
            pl.BlockSpec((_HID, _HID // 2), lambda i: (0, 0)),
            pl.BlockSpec((1, _HID // 2), lambda i: (0, 0)),
            pl.BlockSpec((_HID // 2, 1), lambda i: (0, 0)),
            pl.BlockSpec((1, 1), lambda i: (0, 0)),
        ],
        out_specs=pl.BlockSpec((_NG, 1), lambda i: (0, 0)),
        out_shape=jax.ShapeDtypeStruct((_NG, 1), jnp.float32),
        scratch_shapes=[
            pltpu.VMEM((_NG, _HID), jnp.float32),
            pltpu.VMEM((_NG, 1), jnp.float32),
        ],
    )(z2, w1, b1.reshape(1, _HID), w2, b2.reshape(1, _HID),
      gamma.reshape(1, _HID), beta.reshape(1, _HID), batch_row,
      hw1, hb1.reshape(1, -1), hw2, hb2.reshape(1, -1))


def kernel(x, edge_index, edge_attr, batch, task_id, params):
    src = edge_index[0]
    dst = edge_index[1]
    pad = _NP - _N
    h = jnp.concatenate([x, jnp.zeros((pad, _F_IN), jnp.float32)], axis=0)
    batch_row = jnp.concatenate(
        [batch, jnp.full((pad,), _NG, jnp.int32)]).reshape(1, _NP)
    hsel = jax.tree_util.tree_map(
        lambda a, b: jnp.where(task_id == 0, a, b),
        params['heads'][0], params['heads'][1],
    )
    for i in range(2):
        z2 = _AGG(h, src, dst, jnp.zeros((_NP, h.shape[1]), jnp.float32))
        p = params['gin'][i]
        bn = params['bn'][i]
        h = _mlp_layer(z2, p['w1'], p['b1'], p['w2'],
                       p['b2'], bn['gamma'], bn['beta'])
    z2 = _AGG(h, src, dst, jnp.zeros((_NP, h.shape[1]), jnp.float32))
    p = params['gin'][2]
    bn = params['bn'][2]
    return _mlp3_pool_head(z2, p['w1'], p['b1'], p['w2'],
                           p['b2'], bn['gamma'], bn['beta'], batch_row,
                           hsel['w1'], hsel['b1'], hsel['w2'], hsel['b2'])


# speedup vs baseline: 1.5374x; 1.0223x over previous
import functools

import jax
import jax.numpy as jnp
from jax import lax
from jax.experimental import pallas as pl
from jax.experimental.pallas import tpu as pltpu
from jax.experimental.pallas import tpu_sc as plsc

_N = 10000
_NP = 10240
_E = 320000
_D = 128
_F_IN = 128
_HID = 64
_NG = 64
_BN_EPS = 1e-5

_NC = 2
_NS = 16
_NW = _NC * _NS
_CHUNK = 80
_NCHUNK = _E // _NW // _CHUNK


def _agg_kernel(d):
    per_w = _E // _NW
    rows_per_sub = _NP // _NS
    mesh = plsc.VectorSubcoreMesh(core_axis_name="c", subcore_axis_name="s")
    cp = (None if d == _D
          else pltpu.CompilerParams(use_tc_tiling_on_sc=False))
    _NB = 3 if d == _D else 6
    _ITERS = _NCHUNK // _NB
    _TAIL = _NCHUNK - _ITERS * _NB

    @functools.partial(
        pl.kernel,
        mesh=mesh,
        out_type=jax.ShapeDtypeStruct((_NC, _NP, d), jnp.float32),
        compiler_params=cp,
        scratch_types=(
            [pltpu.VMEM((per_w,), jnp.int32)]
            + [pltpu.VMEM((_CHUNK,), jnp.int32) for _ in range(_NB)]
            + [
                pltpu.VMEM((_NB, _CHUNK, d), jnp.float32),
                pltpu.VMEM_SHARED((_NP, d), jnp.float32),
                pltpu.SemaphoreType.DMA((_NB,)),
                pltpu.SemaphoreType.DMA((_NB,)),
                pltpu.SemaphoreType.DMA((_NB,)),
                pltpu.SemaphoreType.DMA,
            ]
        ),
    )
    def agg(h_hbm, src_hbm, dst_hbm, zeros_hbm, out_hbm, *scr):
        src1d = scr[0]
        dstbs = list(scr[1:1 + _NB])
        rows_v, acc_sh, gsem, ssem, isem, nsem = scr[1 + _NB:]
        cid = lax.axis_index("c")
        sid = lax.axis_index("s")
        wid = sid * _NC + cid
        r0 = sid * rows_per_sub
        base = wid * per_w

        @pl.when(cid == 0)
        def _():
            pltpu.async_copy(h_hbm.at[pl.ds(r0, rows_per_sub)],
                             acc_sh.at[pl.ds(r0, rows_per_sub)], nsem)

        @pl.when(cid != 0)
        def _():
            pltpu.async_copy(zeros_hbm.at[pl.ds(r0, rows_per_sub)],
                             acc_sh.at[pl.ds(r0, rows_per_sub)], nsem)

        pltpu.sync_copy(src_hbm.at[pl.ds(base, per_w)], src1d)

        def it(t, carry):
            ids, gds = [], []
            for b in range(_NB):
                j = t * _NB + b

                @pl.when(t > 0)
                def _(b=b):
                    pltpu.make_async_copy(
                        rows_v.at[b], acc_sh.at[dstbs[b]],
                        ssem.at[b]).wait()

                ids.append(pltpu.async_copy(
                    dst_hbm.at[pl.ds(base + j * _CHUNK, _CHUNK)],
                    dstbs[b], isem.at[b]))
                gds.append(pltpu.async_copy(
                    h_hbm.at[src1d.at[pl.ds(j * _CHUNK, _CHUNK)]],
                    rows_v.at[b], gsem.at[b]))

            @pl.when(t == 0)
            def _():
                pltpu.make_async_copy(
                    zeros_hbm.at[pl.ds(r0, rows_per_sub)],
                    acc_sh.at[pl.ds(r0, rows_per_sub)], nsem).wait()
                plsc.subcore_barrier()

            for b in range(_NB):
                gds[b].wait()
                ids[b].wait()
                pltpu.async_copy(rows_v.at[b], acc_sh.at[dstbs[b]],
                                 ssem.at[b], add=True)
            return carry

        lax.fori_loop(0, _ITERS, it, 0)

        for b in range(_NB):
            pltpu.make_async_copy(rows_v.at[b], acc_sh.at[dstbs[b]],
                                  ssem.at[b]).wait()
        for k in range(_TAIL):
            j = _ITERS * _NB + k
            pltpu.sync_copy(dst_hbm.at[pl.ds(base + j * _CHUNK, _CHUNK)],
                            dstbs[k])
            pltpu.async_copy(
                h_hbm.at[src1d.at[pl.ds(j * _CHUNK, _CHUNK)]],
                rows_v.at[k], gsem.at[k]).wait()
            pltpu.async_copy(rows_v.at[k], acc_sh.at[dstbs[k]],
                             ssem.at[k], add=True)
        for k in range(_TAIL):
            pltpu.make_async_copy(rows_v.at[k], acc_sh.at[dstbs[k]],
                                  ssem.at[k]).wait()

        plsc.subcore_barrier()
        pltpu.sync_copy(acc_sh.at[pl.ds(r0, rows_per_sub)],
                        out_hbm.at[cid, pl.ds(r0, rows_per_sub)])

    return agg


@functools.lru_cache(maxsize=None)
def _agg_cached(d):
    return _agg_kernel(d)


def _AGG(h, src, dst, zeros):
    return _agg_cached(h.shape[1])(h, src, dst, zeros)


def _mlp_layer(z2, w1, b1, w2, b2, gamma, beta):
    din = z2.shape[2]
    blk = 2048
    grid = _NP // blk

    def body(z_ref, w1_ref, b1_ref, w2_ref, b2_ref, g_ref, bt_ref, o_ref):
        z = z_ref[0] + z_ref[1]
        a = jnp.maximum(
            lax.dot(z, w1_ref[...], preferred_element_type=jnp.float32)
            + b1_ref[...], 0.0)
        zz = (lax.dot(a, w2_ref[...], preferred_element_type=jnp.float32)
              + b2_ref[...])
        scale = g_ref[...] * lax.rsqrt(jnp.float32(1.0 + _BN_EPS))
        o_ref[...] = jnp.maximum(zz * scale + bt_ref[...], 0.0)

    return pl.pallas_call(
        body,
        grid=(grid,),
        in_specs=[
            pl.BlockSpec((_NC, blk, din), lambda i: (0, i, 0)),
            pl.BlockSpec((din, _HID), lambda i: (0, 0)),
            pl.BlockSpec((1, _HID), lambda i: (0, 0)),
            pl.BlockSpec((_HID, _HID), lambda i: (0, 0)),
            pl.BlockSpec((1, _HID), lambda i: (0, 0)),
            pl.BlockSpec((1, _HID), lambda i: (0, 0)),
            pl.BlockSpec((1, _HID), lambda i: (0, 0)),
        ],
        out_specs=pl.BlockSpec((blk, _HID), lambda i: (i, 0)),
        out_shape=jax.ShapeDtypeStruct((_NP, _HID), jnp.float32),
    )(z2, w1, b1.reshape(1, _HID), w2, b2.reshape(1, _HID),
      gamma.reshape(1, _HID), beta.reshape(1, _HID))


def _mlp3_pool_head(z2, w1, b1, w2, b2, gamma, beta, batch_row,
                    hw1, hb1, hw2, hb2):
    blk = 2048
    grid = _NP // blk

    def body(z_ref, w1_ref, b1_ref, w2_ref, b2_ref, g_ref, bt_ref,
             b_row_ref, hw1_ref, hb1_ref, hw2_ref, hb2_ref, o_ref,
             acc_ref, cnt_ref):
        i = pl.program_id(0)
        z = z_ref[0] + z_ref[1]
        a = jnp.maximum(
            lax.dot(z, w1_ref[...], preferred_element_type=jnp.float32)
            + b1_ref[...], 0.0)
        zz = (lax.dot(a, w2_ref[...], preferred_element_type=jnp.float32)
              + b2_ref[...])
        scale = g_ref[...] * lax.rsqrt(jnp.float32(1.0 + _BN_EPS))
        res = jnp.maximum(zz * scale + bt_ref[...], 0.0)
        gids = lax.broadcasted_iota(jnp.int32, (_NG, blk), 0)
        oh = (b_row_ref[...] == gids).astype(jnp.float32)
        part = lax.dot(oh, res, preferred_element_type=jnp.float32,
                       precision=lax.Precision.HIGHEST)
        cnt = jnp.sum(oh, axis=1, keepdims=True)

        @pl.when(i == 0)
        def _():
            acc_ref[...] = part
            cnt_ref[...] = cnt

        @pl.when(i > 0)
        def _():
            acc_ref[...] += part
            cnt_ref[...] += cnt

        @pl.when(i == grid - 1)
        def _():
            pooled = acc_ref[...] / jnp.maximum(cnt_ref[...], 1.0)
            ha = jnp.maximum(
                lax.dot(pooled, hw1_ref[...],
                        preferred_element_type=jnp.float32)
                + hb1_ref[...], 0.0)
            o_ref[...] = (lax.dot(ha, hw2_ref[...],
                                  preferred_element_type=jnp.float32)
                          + hb2_ref[...])

    return pl.pallas_call(
        body,
        grid=(grid,),
        in_specs=[
            pl.BlockSpec((_NC, blk, _HID), lambda i: (0, i, 0)),
            pl.BlockSpec((_HID, _HID), lambda i: (0, 0)),
            pl.BlockSpec((1, _HID), lambda i: (0, 0)),
            pl.BlockSpec((_HID, _HID), lambda i: (0, 0)),
            pl.BlockSpec((1, _HID), lambda i: (0, 0)),
            pl.BlockSpec((1, _HID), lambda i: (0, 0)),
            pl.BlockSpec((1, _HID), lambda i: (0, 0)),
            pl.BlockSpec((1, blk), lambda i: (0, i)),
            pl.BlockSpec((_HID, _HID // 2), lambda i: (0, 0)),
            pl.BlockSpec((1, _HID // 2), lambda i: (0, 0)),
            pl.BlockSpec((_HID // 2, 1), lambda i: (0, 0)),
            pl.BlockSpec((1, 1), lambda i: (0, 0)),
        ],
        out_specs=pl.BlockSpec((_NG, 1), lambda i: (0, 0)),
        out_shape=jax.ShapeDtypeStruct((_NG, 1), jnp.float32),
        scratch_shapes=[
            pltpu.VMEM((_NG, _HID), jnp.float32),
            pltpu.VMEM((_NG, 1), jnp.float32),
        ],
    )(z2, w1, b1.reshape(1, _HID), w2, b2.reshape(1, _HID),
      gamma.reshape(1, _HID), beta.reshape(1, _HID), batch_row,
      hw1, hb1.reshape(1, -1), hw2, hb2.reshape(1, -1))


def kernel(x, edge_index, edge_attr, batch, task_id, params):
    src = edge_index[0]
    dst = edge_index[1]
    pad = _NP - _N
    h = jnp.concatenate([x, jnp.zeros((pad, _F_IN), jnp.float32)], axis=0)
    batch_row = jnp.concatenate(
        [batch, jnp.full((pad,), _NG, jnp.int32)]).reshape(1, _NP)
    hsel = jax.tree_util.tree_map(
        lambda a, b: jnp.where(task_id == 0, a, b),
        params['heads'][0], params['heads'][1],
    )
    for i in range(2):
        z2 = _AGG(h, src, dst, jnp.zeros((_NP, h.shape[1]), jnp.float32))
        p = params['gin'][i]
        bn = params['bn'][i]
        h = _mlp_layer(z2, p['w1'], p['b1'], p['w2'],
                       p['b2'], bn['gamma'], bn['beta'])
    z2 = _AGG(h, src, dst, jnp.zeros((_NP, h.shape[1]), jnp.float32))
    p = params['gin'][2]
    bn = params['bn'][2]
    return _mlp3_pool_head(z2, p['w1'], p['b1'], p['w2'],
                           p['b2'], bn['gamma'], bn['beta'], batch_row,
                           hsel['w1'], hsel['b1'], hsel['w2'], hsel['b2'])


# speedup vs baseline: 1.5837x; 1.0301x over previous
import functools

import jax
import jax.numpy as jnp
from jax import lax
from jax.experimental import pallas as pl
from jax.experimental.pallas import tpu as pltpu
from jax.experimental.pallas import tpu_sc as plsc

_N = 10000
_NP = 10240
_E = 320000
_D = 128
_F_IN = 128
_HID = 64
_NG = 64
_BN_EPS = 1e-5

_NC = 2
_NS = 16
_NW = _NC * _NS
_CHUNK = 80
_NCHUNK = _E // _NW // _CHUNK


def _agg_kernel(d):
    per_w = _E // _NW
    rows_per_sub = _NP // _NS
    mesh = plsc.VectorSubcoreMesh(core_axis_name="c", subcore_axis_name="s")
    cp = (None if d == _D
          else pltpu.CompilerParams(use_tc_tiling_on_sc=False))
    if d == _D:
        _NB = 4
        src_chunks = 64
        phases = [(0, 16), (64, 15)]
        tail = [124]
    else:
        _NB = 6
        src_chunks = _NCHUNK
        phases = [(0, 20)]
        tail = [120, 121, 122, 123, 124]

    @functools.partial(
        pl.kernel,
        mesh=mesh,
        out_type=jax.ShapeDtypeStruct((_NC, _NP, d), jnp.float32),
        compiler_params=cp,
        scratch_types=(
            [pltpu.VMEM((src_chunks * _CHUNK,), jnp.int32)]
            + [pltpu.VMEM((_CHUNK,), jnp.int32) for _ in range(_NB)]
            + [
                pltpu.VMEM((_NB, _CHUNK, d), jnp.float32),
                pltpu.VMEM_SHARED((_NP, d), jnp.float32),
                pltpu.SemaphoreType.DMA((_NB,)),
                pltpu.SemaphoreType.DMA((_NB,)),
                pltpu.SemaphoreType.DMA((_NB,)),
                pltpu.SemaphoreType.DMA,
            ]
        ),
    )
    def agg(h_hbm, src_hbm, dst_hbm, zeros_hbm, out_hbm, *scr):
        src1d = scr[0]
        dstbs = list(scr[1:1 + _NB])
        rows_v, acc_sh, gsem, ssem, isem, nsem = scr[1 + _NB:]
        cid = lax.axis_index("c")
        sid = lax.axis_index("s")
        wid = sid * _NC + cid
        r0 = sid * rows_per_sub
        base = wid * per_w

        @pl.when(cid == 0)
        def _():
            pltpu.async_copy(h_hbm.at[pl.ds(r0, rows_per_sub)],
                             acc_sh.at[pl.ds(r0, rows_per_sub)], nsem)

        @pl.when(cid != 0)
        def _():
            pltpu.async_copy(zeros_hbm.at[pl.ds(r0, rows_per_sub)],
                             acc_sh.at[pl.ds(r0, rows_per_sub)], nsem)

        pltpu.sync_copy(
            src_hbm.at[pl.ds(base, src_chunks * _CHUNK)],
            src1d.at[pl.ds(0, src_chunks * _CHUNK)])

        def slot_wait(b):
            pltpu.make_async_copy(rows_v.at[b], acc_sh.at[dstbs[b]],
                                  ssem.at[b]).wait()

        def make_body(chunk0, first):
            def it(t, carry):
                ids, gds = [], []
                for b in range(_NB):
                    j = t * _NB + b

                    if first:
                        @pl.when(t > 0)
                        def _(b=b):
                            slot_wait(b)
                    else:
                        slot_wait(b)

                    ids.append(pltpu.async_copy(
                        dst_hbm.at[pl.ds(
                            base + (chunk0 + j) * _CHUNK, _CHUNK)],
                        dstbs[b], isem.at[b]))
                    gds.append(pltpu.async_copy(
                        h_hbm.at[src1d.at[pl.ds(j * _CHUNK, _CHUNK)]],
                        rows_v.at[b], gsem.at[b]))

                if first:
                    @pl.when(t == 0)
                    def _():
                        pltpu.make_async_copy(
                            zeros_hbm.at[pl.ds(r0, rows_per_sub)],
                            acc_sh.at[pl.ds(r0, rows_per_sub)],
                            nsem).wait()
                        plsc.subcore_barrier()

                for b in range(_NB):
                    gds[b].wait()
                    ids[b].wait()
                    pltpu.async_copy(rows_v.at[b], acc_sh.at[dstbs[b]],
                                     ssem.at[b], add=True)
                return carry
            return it

        for pi, (chunk0, iters) in enumerate(phases):
            if pi > 0:
                n_left = (_NCHUNK - chunk0) * _CHUNK
                pltpu.sync_copy(
                    src_hbm.at[pl.ds(base + chunk0 * _CHUNK, n_left)],
                    src1d.at[pl.ds(0, n_left)])
            lax.fori_loop(0, iters, make_body(chunk0, pi == 0), 0)

        for b in range(_NB):
            slot_wait(b)
        last_chunk0 = phases[-1][0]
        for k, j in enumerate(tail):
            pltpu.sync_copy(dst_hbm.at[pl.ds(base + j * _CHUNK, _CHUNK)],
                            dstbs[k])
            pltpu.async_copy(
                h_hbm.at[src1d.at[pl.ds((j - last_chunk0) * _CHUNK,
                                        _CHUNK)]],
                rows_v.at[k], gsem.at[k]).wait()
            pltpu.async_copy(rows_v.at[k], acc_sh.at[dstbs[k]],
                             ssem.at[k], add=True)
        for k in range(len(tail)):
            pltpu.make_async_copy(rows_v.at[k], acc_sh.at[dstbs[k]],
                                  ssem.at[k]).wait()

        plsc.subcore_barrier()
        pltpu.sync_copy(acc_sh.at[pl.ds(r0, rows_per_sub)],
                        out_hbm.at[cid, pl.ds(r0, rows_per_sub)])

    return agg


@functools.lru_cache(maxsize=None)
def _agg_cached(d):
    return _agg_kernel(d)


def _AGG(h, src, dst, zeros):
    return _agg_cached(h.shape[1])(h, src, dst, zeros)


def _mlp_layer(z2, w1, b1, w2, b2, gamma, beta):
    din = z2.shape[2]
    blk = 2048
    grid = _NP // blk

    def body(z_ref, w1_ref, b1_ref, w2_ref, b2_ref, g_ref, bt_ref, o_ref):
        z = z_ref[0] + z_ref[1]
        a = jnp.maximum(
            lax.dot(z, w1_ref[...], preferred_element_type=jnp.float32)
            + b1_ref[...], 0.0)
        zz = (lax.dot(a, w2_ref[...], preferred_element_type=jnp.float32)
              + b2_ref[...])
        scale = g_ref[...] * lax.rsqrt(jnp.float32(1.0 + _BN_EPS))
        o_ref[...] = jnp.maximum(zz * scale + bt_ref[...], 0.0)

    return pl.pallas_call(
        body,
        grid=(grid,),
        in_specs=[
            pl.BlockSpec((_NC, blk, din), lambda i: (0, i, 0)),
            pl.BlockSpec((din, _HID), lambda i: (0, 0)),
            pl.BlockSpec((1, _HID), lambda i: (0, 0)),
            pl.BlockSpec((_HID, _HID), lambda i: (0, 0)),
            pl.BlockSpec((1, _HID), lambda i: (0, 0)),
            pl.BlockSpec((1, _HID), lambda i: (0, 0)),
            pl.BlockSpec((1, _HID), lambda i: (0, 0)),
        ],
        out_specs=pl.BlockSpec((blk, _HID), lambda i: (i, 0)),
        out_shape=jax.ShapeDtypeStruct((_NP, _HID), jnp.float32),
    )(z2, w1, b1.reshape(1, _HID), w2, b2.reshape(1, _HID),
      gamma.reshape(1, _HID), beta.reshape(1, _HID))


def _mlp3_pool_head(z2, w1, b1, w2, b2, gamma, beta, batch_row,
                    hw1, hb1, hw2, hb2):
    blk = 2048
    grid = _NP // blk

    def body(z_ref, w1_ref, b1_ref, w2_ref, b2_ref, g_ref, bt_ref,
             b_row_ref, hw1_ref, hb1_ref, hw2_ref, hb2_ref, o_ref,
             acc_ref, cnt_ref):
        i = pl.program_id(0)
        z = z_ref[0] + z_ref[1]
        a = jnp.maximum(
            lax.dot(z, w1_ref[...], preferred_element_type=jnp.float32)
            + b1_ref[...], 0.0)
        zz = (lax.dot(a, w2_ref[...], preferred_element_type=jnp.float32)
              + b2_ref[...])
        scale = g_ref[...] * lax.rsqrt(jnp.float32(1.0 + _BN_EPS))
        res = jnp.maximum(zz * scale + bt_ref[...], 0.0)
        gids = lax.broadcasted_iota(jnp.int32, (_NG, blk), 0)
        oh = (b_row_ref[...] == gids).astype(jnp.float32)
        part = lax.dot(oh, res, preferred_element_type=jnp.float32,
                       precision=lax.Precision.HIGHEST)
        cnt = jnp.sum(oh, axis=1, keepdims=True)

        @pl.when(i == 0)
        def _():
            acc_ref[...] = part
            cnt_ref[...] = cnt

        @pl.when(i > 0)
        def _():
            acc_ref[...] += part
            cnt_ref[...] += cnt

        @pl.when(i == grid - 1)
        def _():
            pooled = acc_ref[...] / jnp.maximum(cnt_ref[...], 1.0)
            ha = jnp.maximum(
                lax.dot(pooled, hw1_ref[...],
                        preferred_element_type=jnp.float32)
                + hb1_ref[...], 0.0)
            o_ref[...] = (lax.dot(ha, hw2_ref[...],
                                  preferred_element_type=jnp.float32)
                          + hb2_ref[...])

    return pl.pallas_call(
        body,
        grid=(grid,),
        in_specs=[
            pl.BlockSpec((_NC, blk, _HID), lambda i: (0, i, 0)),
            pl.BlockSpec((_HID, _HID), lambda i: (0, 0)),
            pl.BlockSpec((1, _HID), lambda i: (0, 0)),
            pl.BlockSpec((_HID, _HID), lambda i: (0, 0)),
            pl.BlockSpec((1, _HID), lambda i: (0, 0)),
            pl.BlockSpec((1, _HID), lambda i: (0, 0)),
            pl.BlockSpec((1, _HID), lambda i: (0, 0)),
            pl.BlockSpec((1, blk), lambda i: (0, i)),
            pl.BlockSpec((_HID, _HID // 2), lambda i: (0, 0)),
            pl.BlockSpec((1, _HID // 2), lambda i: (0, 0)),
            pl.BlockSpec((_HID // 2, 1), lambda i: (0, 0)),
            pl.BlockSpec((1, 1), lambda i: (0, 0)),
        ],
        out_specs=pl.BlockSpec((_NG, 1), lambda i: (0, 0)),
        out_shape=jax.ShapeDtypeStruct((_NG, 1), jnp.float32),
        scratch_shapes=[
            pltpu.VMEM((_NG, _HID), jnp.float32),
            pltpu.VMEM((_NG, 1), jnp.float32),
        ],
    )(z2, w1, b1.reshape(1, _HID), w2, b2.reshape(1, _HID),
      gamma.reshape(1, _HID), beta.reshape(1, _HID), batch_row,
      hw1, hb1.reshape(1, -1), hw2, hb2.reshape(1, -1))


def kernel(x, edge_index, edge_attr, batch, task_id, params):
    src = edge_index[0]
    dst = edge_index[1]
    pad = _NP - _N
    h = jnp.concatenate([x, jnp.zeros((pad, _F_IN), jnp.float32)], axis=0)
    batch_row = jnp.concatenate(
        [batch, jnp.full((pad,), _NG, jnp.int32)]).reshape(1, _NP)
    hsel = jax.tree_util.tree_map(
        lambda a, b: jnp.where(task_id == 0, a, b),
        params['heads'][0], params['heads'][1],
    )
    for i in range(2):
        z2 = _AGG(h, src, dst, jnp.zeros((_NP, h.shape[1]), jnp.float32))
        p = params['gin'][i]
        bn = params['bn'][i]
        h = _mlp_layer(z2, p['w1'], p['b1'], p['w2'],
                       p['b2'], bn['gamma'], bn['beta'])
    z2 = _AGG(h, src, dst, jnp.zeros((_NP, h.shape[1]), jnp.float32))
    p = params['gin'][2]
    bn = params['bn'][2]
    return _mlp3_pool_head(z2, p['w1'], p['b1'], p['w2'],
                           p['b2'], bn['gamma'], bn['beta'], batch_row,
                           hsel['w1'], hsel['b1'], hsel['w2'], hsel['b2'])


# speedup vs baseline: 1.5922x; 1.0054x over previous
import functools

import jax
import jax.numpy as jnp
from jax import lax
from jax.experimental import pallas as pl
from jax.experimental.pallas import tpu as pltpu
from jax.experimental.pallas import tpu_sc as plsc

_N = 10000
_NP = 10240
_E = 320000
_D = 128
_F_IN = 128
_HID = 64
_NG = 64
_BN_EPS = 1e-5

_NC = 2
_NS = 16
_NW = _NC * _NS
_CHUNK = 80
_NCHUNK = _E // _NW // _CHUNK


def _agg_kernel(d):
    per_w = _E // _NW
    rows_per_sub = _NP // _NS
    mesh = plsc.VectorSubcoreMesh(core_axis_name="c", subcore_axis_name="s")
    cp = (None if d == _D
          else pltpu.CompilerParams(use_tc_tiling_on_sc=False))
    if d == _D:
        _NB = 4
        src_chunks = 64
        phases = [(0, 16), (64, 15)]
        tail = [124]
    else:
        _NB = 8
        src_chunks = _NCHUNK
        phases = [(0, 15)]
        tail = [120, 121, 122, 123, 124]

    @functools.partial(
        pl.kernel,
        mesh=mesh,
        out_type=jax.ShapeDtypeStruct((_NC, _NP, d), jnp.float32),
        compiler_params=cp,
        scratch_types=(
            [pltpu.VMEM((src_chunks * _CHUNK,), jnp.int32)]
            + [pltpu.VMEM((_CHUNK,), jnp.int32) for _ in range(_NB)]
            + [
                pltpu.VMEM((_NB, _CHUNK, d), jnp.float32),
                pltpu.VMEM_SHARED((_NP, d), jnp.float32),
                pltpu.SemaphoreType.DMA((_NB,)),
                pltpu.SemaphoreType.DMA((_NB,)),
                pltpu.SemaphoreType.DMA((_NB,)),
                pltpu.SemaphoreType.DMA,
            ]
        ),
    )
    def agg(h_hbm, src_hbm, dst_hbm, zeros_hbm, out_hbm, *scr):
        src1d = scr[0]
        dstbs = list(scr[1:1 + _NB])
        rows_v, acc_sh, gsem, ssem, isem, nsem = scr[1 + _NB:]
        cid = lax.axis_index("c")
        sid = lax.axis_index("s")
        wid = sid * _NC + cid
        r0 = sid * rows_per_sub
        base = wid * per_w

        @pl.when(cid == 0)
        def _():
            pltpu.async_copy(h_hbm.at[pl.ds(r0, rows_per_sub)],
                             acc_sh.at[pl.ds(r0, rows_per_sub)], nsem)

        @pl.when(cid != 0)
        def _():
            pltpu.async_copy(zeros_hbm.at[pl.ds(r0, rows_per_sub)],
                             acc_sh.at[pl.ds(r0, rows_per_sub)], nsem)

        pltpu.sync_copy(
            src_hbm.at[pl.ds(base, src_chunks * _CHUNK)],
            src1d.at[pl.ds(0, src_chunks * _CHUNK)])

        def slot_wait(b):
            pltpu.make_async_copy(rows_v.at[b], acc_sh.at[dstbs[b]],
                                  ssem.at[b]).wait()

        def make_body(chunk0, first):
            def it(t, carry):
                ids, gds = [], []
                for b in range(_NB):
                    j = t * _NB + b

                    if first:
                        @pl.when(t > 0)
                        def _(b=b):
                            slot_wait(b)
                    else:
                        slot_wait(b)

                    ids.append(pltpu.async_copy(
                        dst_hbm.at[pl.ds(
                            base + (chunk0 + j) * _CHUNK, _CHUNK)],
                        dstbs[b], isem.at[b]))
                    gds.append(pltpu.async_copy(
                        h_hbm.at[src1d.at[pl.ds(j * _CHUNK, _CHUNK)]],
                        rows_v.at[b], gsem.at[b]))

                if first:
                    @pl.when(t == 0)
                    def _():
                        pltpu.make_async_copy(
                            zeros_hbm.at[pl.ds(r0, rows_per_sub)],
                            acc_sh.at[pl.ds(r0, rows_per_sub)],
                            nsem).wait()
                        plsc.subcore_barrier()

                for b in range(_NB):
                    gds[b].wait()
                    ids[b].wait()
                    pltpu.async_copy(rows_v.at[b], acc_sh.at[dstbs[b]],
                                     ssem.at[b], add=True)
                return carry
            return it

        for pi, (chunk0, iters) in enumerate(phases):
            if pi > 0:
                n_left = (_NCHUNK - chunk0) * _CHUNK
                pltpu.sync_copy(
                    src_hbm.at[pl.ds(base + chunk0 * _CHUNK, n_left)],
                    src1d.at[pl.ds(0, n_left)])
            lax.fori_loop(0, iters, make_body(chunk0, pi == 0), 0)

        for b in range(_NB):
            slot_wait(b)
        last_chunk0 = phases[-1][0]
        for k, j in enumerate(tail):
            pltpu.sync_copy(dst_hbm.at[pl.ds(base + j * _CHUNK, _CHUNK)],
                            dstbs[k])
            pltpu.async_copy(
                h_hbm.at[src1d.at[pl.ds((j - last_chunk0) * _CHUNK,
                                        _CHUNK)]],
                rows_v.at[k], gsem.at[k]).wait()
            pltpu.async_copy(rows_v.at[k], acc_sh.at[dstbs[k]],
                             ssem.at[k], add=True)
        for k in range(len(tail)):
            pltpu.make_async_copy(rows_v.at[k], acc_sh.at[dstbs[k]],
                                  ssem.at[k]).wait()

        plsc.subcore_barrier()
        pltpu.sync_copy(acc_sh.at[pl.ds(r0, rows_per_sub)],
                        out_hbm.at[cid, pl.ds(r0, rows_per_sub)])

    return agg


@functools.lru_cache(maxsize=None)
def _agg_cached(d):
    return _agg_kernel(d)


def _AGG(h, src, dst, zeros):
    return _agg_cached(h.shape[1])(h, src, dst, zeros)


def _mlp_layer(z2, w1, b1, w2, b2, gamma, beta):
    din = z2.shape[2]
    blk = 2048
    grid = _NP // blk

    def body(z_ref, w1_ref, b1_ref, w2_ref, b2_ref, g_ref, bt_ref, o_ref):
        z = z_ref[0] + z_ref[1]
        a = jnp.maximum(
            lax.dot(z, w1_ref[...], preferred_element_type=jnp.float32)
            + b1_ref[...], 0.0)
        zz = (lax.dot(a, w2_ref[...], preferred_element_type=jnp.float32)
              + b2_ref[...])
        scale = g_ref[...] * lax.rsqrt(jnp.float32(1.0 + _BN_EPS))
        o_ref[...] = jnp.maximum(zz * scale + bt_ref[...], 0.0)

    return pl.pallas_call(
        body,
        grid=(grid,),
        in_specs=[
            pl.BlockSpec((_NC, blk, din), lambda i: (0, i, 0)),
            pl.BlockSpec((din, _HID), lambda i: (0, 0)),
            pl.BlockSpec((1, _HID), lambda i: (0, 0)),
            pl.BlockSpec((_HID, _HID), lambda i: (0, 0)),
            pl.BlockSpec((1, _HID), lambda i: (0, 0)),
            pl.BlockSpec((1, _HID), lambda i: (0, 0)),
            pl.BlockSpec((1, _HID), lambda i: (0, 0)),
        ],
        out_specs=pl.BlockSpec((blk, _HID), lambda i: (i, 0)),
        out_shape=jax.ShapeDtypeStruct((_NP, _HID), jnp.float32),
    )(z2, w1, b1.reshape(1, _HID), w2, b2.reshape(1, _HID),
      gamma.reshape(1, _HID), beta.reshape(1, _HID))


def _mlp3_pool_head(z2, w1, b1, w2, b2, gamma, beta, batch_row,
                    hw1, hb1, hw2, hb2):
    blk = 2048
    grid = _NP // blk

    def body(z_ref, w1_ref, b1_ref, w2_ref, b2_ref, g_ref, bt_ref,
             b_row_ref, hw1_ref, hb1_ref, hw2_ref, hb2_ref, o_ref,
             acc_ref, cnt_ref):
        i = pl.program_id(0)
        z = z_ref[0] + z_ref[1]
        a = jnp.maximum(
            lax.dot(z, w1_ref[...], preferred_element_type=jnp.float32)
            + b1_ref[...], 0.0)
        zz = (lax.dot(a, w2_ref[...], preferred_element_type=jnp.float32)
              + b2_ref[...])
        scale = g_ref[...] * lax.rsqrt(jnp.float32(1.0 + _BN_EPS))
        res = jnp.maximum(zz * scale + bt_ref[...], 0.0)
        gids = lax.broadcasted_iota(jnp.int32, (_NG, blk), 0)
        oh = (b_row_ref[...] == gids).astype(jnp.float32)
        part = lax.dot(oh, res, preferred_element_type=jnp.float32,
                       precision=lax.Precision.HIGHEST)
        cnt = jnp.sum(oh, axis=1, keepdims=True)

        @pl.when(i == 0)
        def _():
            acc_ref[...] = part
            cnt_ref[...] = cnt

        @pl.when(i > 0)
        def _():
            acc_ref[...] += part
            cnt_ref[...] += cnt

        @pl.when(i == grid - 1)
        def _():
            pooled = acc_ref[...] / jnp.maximum(cnt_ref[...], 1.0)
            ha = jnp.maximum(
                lax.dot(pooled, hw1_ref[...],
                        preferred_element_type=jnp.float32)
                + hb1_ref[...], 0.0)
            o_ref[...] = (lax.dot(ha, hw2_ref[...],
                                  preferred_element_type=jnp.float32)
                          + hb2_ref[...])

    return pl.pallas_call(
        body,
        grid=(grid,),
        in_specs=[
            pl.BlockSpec((_NC, blk, _HID), lambda i: (0, i, 0)),
            pl.BlockSpec((_HID, _HID), lambda i: (0, 0)),
            pl.BlockSpec((1, _HID), lambda i: (0, 0)),
            pl.BlockSpec((_HID, _HID), lambda i: (0, 0)),
            pl.BlockSpec((1, _HID), lambda i: (0, 0)),
            pl.BlockSpec((1, _HID), lambda i: (0, 0)),
            pl.BlockSpec((1, _HID), lambda i: (0, 0)),
            pl.BlockSpec((1, blk), lambda i: (0, i)),
            pl.BlockSpec((_HID, _HID // 2), lambda i: (0, 0)),
            pl.BlockSpec((1, _HID // 2), lambda i: (0, 0)),
            pl.BlockSpec((_HID // 2, 1), lambda i: (0, 0)),
            pl.BlockSpec((1, 1), lambda i: (0, 0)),
        ],
        out_specs=pl.BlockSpec((_NG, 1), lambda i: (0, 0)),
        out_shape=jax.ShapeDtypeStruct((_NG, 1), jnp.float32),
        scratch_shapes=[
            pltpu.VMEM((_NG, _HID), jnp.float32),
            pltpu.VMEM((_NG, 1), jnp.float32),
        ],
    )(z2, w1, b1.reshape(1, _HID), w2, b2.reshape(1, _HID),
      gamma.reshape(1, _HID), beta.reshape(1, _HID), batch_row,
      hw1, hb1.reshape(1, -1), hw2, hb2.reshape(1, -1))


def kernel(x, edge_index, edge_attr, batch, task_id, params):
    src = edge_index[0]
    dst = edge_index[1]
    pad = _NP - _N
    h = jnp.concatenate([x, jnp.zeros((pad, _F_IN), jnp.float32)], axis=0)
    batch_row = jnp.concatenate(
        [batch, jnp.full((pad,), _NG, jnp.int32)]).reshape(1, _NP)
    hsel = jax.tree_util.tree_map(
        lambda a, b: jnp.where(task_id == 0, a, b),
        params['heads'][0], params['heads'][1],
    )
    for i in range(2):
        z2 = _AGG(h, src, dst, jnp.zeros((_NP, h.shape[1]), jnp.float32))
        p = params['gin'][i]
        bn = params['bn'][i]
        h = _mlp_layer(z2, p['w1'], p['b1'], p['w2'],
                       p['b2'], bn['gamma'], bn['beta'])
    z2 = _AGG(h, src, dst, jnp.zeros((_NP, h.shape[1]), jnp.float32))
    p = params['gin'][2]
    bn = params['bn'][2]
    return _mlp3_pool_head(z2, p['w1'], p['b1'], p['w2'],
                           p['b2'], bn['gamma'], bn['beta'], batch_row,
                           hsel['w1'], hsel['b1'], hsel['w2'], hsel['b2'])
